# baseline (speedup 1.0000x reference)
import functools

import jax
import jax.numpy as jnp
from jax import lax
from jax.experimental import pallas as pl
from jax.experimental.pallas import tpu as pltpu
from jax.experimental.pallas import tpu_sc as plsc

_NC, _NS = 2, 16
_NW = _NC * _NS
_BLK = 128


def _sc_seg_sum_body(aidx_hbm, gidx_hbm, xa_hbm, cond_hbm, z80, z64, z1,
                     xsum_out, csum_out, cnt_out,
                     aidx, gidx, xrows, crows, ones,
                     xsum_sh, csum_sh, cnt_sh, sem1, sem2):
    c = lax.axis_index("c")
    s = lax.axis_index("s")
    wid = s * _NC + c
    Gp = cnt_sh.shape[0]
    ninc = aidx_hbm.shape[0]
    nblk_tot = ninc // _BLK

    for i in range(_BLK // 16):
        ones[pl.ds(16 * i, 16)] = jnp.ones((16,), jnp.float32)

    rows_per_s = Gp // _NS
    r0 = s * rows_per_s
    pltpu.sync_copy(z80.at[pl.ds(r0, rows_per_s)], xsum_sh.at[pl.ds(r0, rows_per_s)])
    pltpu.sync_copy(z64.at[pl.ds(r0, rows_per_s)], csum_sh.at[pl.ds(r0, rows_per_s)])
    pltpu.sync_copy(z1.at[pl.ds(r0, rows_per_s)], cnt_sh.at[pl.ds(r0, rows_per_s)])
    plsc.subcore_barrier()

    nblk_w = (nblk_tot - wid + _NW - 1) // _NW

    def body(i, carry):
        b = wid + i * _NW
        base = b * _BLK
        pltpu.sync_copy(aidx_hbm.at[pl.ds(base, _BLK)], aidx)
        pltpu.sync_copy(gidx_hbm.at[pl.ds(base, _BLK)], gidx)
        pltpu.async_copy(xa_hbm.at[aidx], xrows, sem1).wait()
        pltpu.async_copy(cond_hbm.at[aidx], crows, sem2).wait()
        pltpu.sync_copy(xrows, xsum_sh.at[gidx], add=True)
        pltpu.sync_copy(crows, csum_sh.at[gidx], add=True)
        pltpu.sync_copy(ones, cnt_sh.at[gidx], add=True)
        return carry

    lax.fori_loop(0, nblk_w, body, 0)
    plsc.subcore_barrier()

    pltpu.sync_copy(xsum_sh.at[pl.ds(r0, rows_per_s)],
                    xsum_out.at[c, pl.ds(r0, rows_per_s)])
    pltpu.sync_copy(csum_sh.at[pl.ds(r0, rows_per_s)],
                    csum_out.at[c, pl.ds(r0, rows_per_s)])
    pltpu.sync_copy(cnt_sh.at[pl.ds(r0, rows_per_s)],
                    cnt_out.at[c, pl.ds(r0, rows_per_s)])


def _mm(x, W):
    return lax.dot_general(x, W, (((1,), (1,)), ((), ())),
                           preferred_element_type=jnp.float32)


def _sigmoid(x):
    return 1.0 / (1.0 + jnp.exp(-x))


def _tc_mid_body(xs_ref, cs_ref, cn_ref, Wih_ref, Whh_ref, bih_ref, bhh_ref,
                 Wg1_ref, bg1_ref, Wg2_ref, bg2_ref,
                 Wb1_ref, bb1_ref, Wb2_ref, bb2_ref,
                 q2_ref, cond_ref, gamma_ref, beta_ref, cnt_ref):
    Gm = q2_ref.shape[0]
    D = q2_ref.shape[1]
    cnt = cn_ref[0, :Gm] + cn_ref[1, :Gm]
    xsum = xs_ref[0, :Gm] + xs_ref[1, :Gm]
    csum = cs_ref[0, :Gm] + cs_ref[1, :Gm]
    cntc = cnt[:, None]
    r1 = xsum / (cntc + 1e-16)
    cond_g = csum / jnp.maximum(cntc, 1.0)

    bihv = bih_ref[...]
    bhhv = bhh_ref[...]
    gates1 = bihv + bhhv
    i1 = _sigmoid(gates1[:, 0:D])
    g1 = jnp.tanh(gates1[:, 2 * D:3 * D])
    o1 = _sigmoid(gates1[:, 3 * D:4 * D])
    c1 = i1 * g1
    h1 = o1 * jnp.tanh(c1)
    h1b = jnp.broadcast_to(h1, (Gm, D))
    q_star1 = jnp.concatenate([h1b, r1], axis=1)

    gates = _mm(q_star1, Wih_ref[...]) + bihv + _mm(h1b, Whh_ref[...]) + bhhv
    i2 = _sigmoid(gates[:, 0:D])
    f2 = _sigmoid(gates[:, D:2 * D])
    g2 = jnp.tanh(gates[:, 2 * D:3 * D])
    o2 = _sigmoid(gates[:, 3 * D:4 * D])
    c2 = f2 * jnp.broadcast_to(c1, (Gm, D)) + i2 * g2
    q2_ref[...] = o2 * jnp.tanh(c2)

    cond_ref[...] = cond_g
    gamma_ref[...] = _mm(jnp.maximum(_mm(cond_g, Wg1_ref[...]) + bg1_ref[...],
                                     0.0), Wg2_ref[...]) + bg2_ref[...]
    beta_ref[...] = _mm(jnp.maximum(_mm(cond_g, Wb1_ref[...]) + bb1_ref[...],
                                    0.0), Wb2_ref[...]) + bb2_ref[...]
    cnt_ref[...] = cntc


def _tc_mid(xs, cs, cn, Wih, Whh, bih, bhh, Wg1, bg1, Wg2, bg2,
            Wb1, bb1, Wb2, bb2, Gm):
    D = 80
    Dc = cs.shape[2]
    return pl.pallas_call(
        _tc_mid_body,
        out_shape=(
            jax.ShapeDtypeStruct((Gm, D), jnp.float32),
            jax.ShapeDtypeStruct((Gm, Dc), jnp.float32),
            jax.ShapeDtypeStruct((Gm, 199), jnp.float32),
            jax.ShapeDtypeStruct((Gm, 199), jnp.float32),
            jax.ShapeDtypeStruct((Gm, 1), jnp.float32),
        ),
    )(xs, cs, cn, Wih, Whh, bih.reshape(1, -1), bhh.reshape(1, -1),
      Wg1, bg1.reshape(1, -1), Wg2, bg2.reshape(1, -1),
      Wb1, bb1.reshape(1, -1), Wb2, bb2.reshape(1, -1))


def _tc_fin_body(xg0_ref, q2_ref, r2_ref, den_ref, cnt_ref, gamma_ref,
                 beta_ref, cond_ref, Wm_ref, bm_ref, out_ref):
    Gm = xg0_ref.shape[0]
    r2 = ((r2_ref[0, :Gm] + r2_ref[1, :Gm])
          / (den_ref[0, :Gm] + den_ref[1, :Gm] + 1e-16)[:, None])
    present = cnt_ref[...] > 0.0
    q_star = jnp.where(present, jnp.concatenate([q2_ref[...], r2], axis=1), 0.0)
    xg = _mm(jnp.concatenate([xg0_ref[...], q_star], axis=1), Wm_ref[...]) \
        + bm_ref[...]
    xg = gamma_ref[...] * xg + beta_ref[...]
    out_ref[...] = jnp.concatenate([xg, cond_ref[...]], axis=1)


def _tc_fin(xg0, q2, r2p, denp, cnt, gamma, beta, cond_g, W_merge, b_merge):
    Gm = xg0.shape[0]
    return pl.pallas_call(
        _tc_fin_body,
        out_shape=jax.ShapeDtypeStruct((Gm, 263), jnp.float32),
    )(xg0, q2, r2p, denp, cnt, gamma, beta, cond_g, W_merge,
      b_merge.reshape(1, -1))


def _sc_attn_body(aidx_hbm, gidx_hbm, xa_hbm, q2_hbm, z80, z1,
                  r2_out, den_out,
                  aidx, gidx, xrows, qrows, srows, wblk,
                  r2_sh, den_sh, sem1, sem2):
    c = lax.axis_index("c")
    s = lax.axis_index("s")
    wid = s * _NC + c
    Gp = den_sh.shape[0]
    ninc = aidx_hbm.shape[0]
    nblk_tot = ninc // _BLK
    D = xrows.shape[1]
    nch = D // 16

    rows_per_s = Gp // _NS
    r0 = s * rows_per_s
    pltpu.sync_copy(z80.at[pl.ds(r0, rows_per_s)], r2_sh.at[pl.ds(r0, rows_per_s)])
    pltpu.sync_copy(z1.at[pl.ds(r0, rows_per_s)], den_sh.at[pl.ds(r0, rows_per_s)])
    plsc.subcore_barrier()

    iota16 = lax.iota(jnp.int32, 16)
    nblk_w = (nblk_tot - wid + _NW - 1) // _NW

    def block_body(bi, carry):
        b = wid + bi * _NW
        base = b * _BLK
        pltpu.sync_copy(aidx_hbm.at[pl.ds(base, _BLK)], aidx)
        pltpu.sync_copy(gidx_hbm.at[pl.ds(base, _BLK)], gidx)
        pltpu.async_copy(xa_hbm.at[aidx], xrows, sem1).wait()
        pltpu.async_copy(q2_hbm.at[gidx], qrows, sem2).wait()

        def tile_body(j, carry2):
            evec = jnp.zeros((16,), jnp.float32)
            for i in range(16):
                row = j * 16 + i
                p = xrows[row, pl.ds(0, 16)] * qrows[row, pl.ds(0, 16)]
                for ch in range(1, nch):
                    p = p + (xrows[row, pl.ds(16 * ch, 16)]
                             * qrows[row, pl.ds(16 * ch, 16)])
                es = jnp.sum(p)
                evec = jnp.where(iota16 == i, jnp.full((16,), es), evec)
            wv = jnp.exp(evec)
            wblk[pl.ds(j * 16, 16)] = wv
            for i in range(16):
                row = j * 16 + i
                ws = wv[i]
                for ch in range(nch):
                    srows[row, pl.ds(16 * ch, 16)] = (
                        xrows[row, pl.ds(16 * ch, 16)] * ws)
            return carry2

        lax.fori_loop(0, _BLK // 16, tile_body, 0)
        pltpu.sync_copy(srows, r2_sh.at[gidx], add=True)
        pltpu.sync_copy(wblk, den_sh.at[gidx], add=True)
        return carry

    lax.fori_loop(0, nblk_w, block_body, 0)
    plsc.subcore_barrier()

    pltpu.sync_copy(r2_sh.at[pl.ds(r0, rows_per_s)],
                    r2_out.at[c, pl.ds(r0, rows_per_s)])
    pltpu.sync_copy(den_sh.at[pl.ds(r0, rows_per_s)],
                    den_out.at[c, pl.ds(r0, rows_per_s)])


def _sc_attn(atom_idx, group_idx, xa_proj, q2, Gm):
    Dx = xa_proj.shape[1]
    Gp = ((Gm + _NS * 8 - 1) // (_NS * 8)) * (_NS * 8)
    z80 = jnp.zeros((Gp, Dx), jnp.float32)
    z1 = jnp.zeros((Gp,), jnp.float32)
    f = pl.kernel(
        _sc_attn_body,
        out_type=(
            jax.ShapeDtypeStruct((2, Gp, Dx), jnp.float32),
            jax.ShapeDtypeStruct((2, Gp), jnp.float32),
        ),
        mesh=plsc.VectorSubcoreMesh(core_axis_name="c", subcore_axis_name="s"),
        compiler_params=pltpu.CompilerParams(use_tc_tiling_on_sc=False,
                                             needs_layout_passes=False),
        scratch_types=[
            pltpu.VMEM((_BLK,), jnp.int32),
            pltpu.VMEM((_BLK,), jnp.int32),
            pltpu.VMEM((_BLK, Dx), jnp.float32),
            pltpu.VMEM((_BLK, Dx), jnp.float32),
            pltpu.VMEM((_BLK, Dx), jnp.float32),
            pltpu.VMEM((_BLK,), jnp.float32),
            pltpu.VMEM_SHARED((Gp, Dx), jnp.float32),
            pltpu.VMEM_SHARED((Gp,), jnp.float32),
            pltpu.SemaphoreType.DMA,
            pltpu.SemaphoreType.DMA,
        ],
    )
    return f(atom_idx, group_idx, xa_proj, q2, z80, z1)


def _sc_seg_sum(atom_idx, group_idx, xa_proj, cond_atom, Gm):
    Dx = xa_proj.shape[1]
    Dc = cond_atom.shape[1]
    Gp = ((Gm + _NS * 8 - 1) // (_NS * 8)) * (_NS * 8)
    z80 = jnp.zeros((Gp, Dx), jnp.float32)
    z64 = jnp.zeros((Gp, Dc), jnp.float32)
    z1 = jnp.zeros((Gp,), jnp.float32)
    f = pl.kernel(
        _sc_seg_sum_body,
        out_type=(
            jax.ShapeDtypeStruct((2, Gp, Dx), jnp.float32),
            jax.ShapeDtypeStruct((2, Gp, Dc), jnp.float32),
            jax.ShapeDtypeStruct((2, Gp), jnp.float32),
        ),
        mesh=plsc.VectorSubcoreMesh(core_axis_name="c", subcore_axis_name="s"),
        compiler_params=pltpu.CompilerParams(use_tc_tiling_on_sc=False),
        scratch_types=[
            pltpu.VMEM((_BLK,), jnp.int32),
            pltpu.VMEM((_BLK,), jnp.int32),
            pltpu.VMEM((_BLK, Dx), jnp.float32),
            pltpu.VMEM((_BLK, Dc), jnp.float32),
            pltpu.VMEM((_BLK,), jnp.float32),
            pltpu.VMEM_SHARED((Gp, Dx), jnp.float32),
            pltpu.VMEM_SHARED((Gp, Dc), jnp.float32),
            pltpu.VMEM_SHARED((Gp,), jnp.float32),
            pltpu.SemaphoreType.DMA,
            pltpu.SemaphoreType.DMA,
        ],
    )
    return f(atom_idx, group_idx, xa_proj, cond_atom, z80, z64, z1)


def _tc_pre_body(x_atom_ref, W_a_ref, b_a_ref, x_group_ref, W_g_ref, b_g_ref,
                 xa_ref, xg0_ref, tid_ref):
    xa_ref[...] = lax.dot_general(
        x_atom_ref[...], W_a_ref[...], (((1,), (1,)), ((), ())),
        preferred_element_type=jnp.float32) + b_a_ref[...]
    Xg = x_group_ref[...][:, 0:40]
    xg0_ref[...] = lax.dot_general(
        Xg, W_g_ref[...], (((1,), (1,)), ((), ())),
        preferred_element_type=jnp.float32) + b_g_ref[...]
    m = jnp.max(Xg, axis=1, keepdims=True)
    cols = lax.broadcasted_iota(jnp.int32, Xg.shape, 1)
    tid = jnp.min(jnp.where(Xg == m, cols, jnp.int32(2**30)), axis=1)
    tid_ref[...] = tid.astype(jnp.int32)


def _tc_pre(x_atom, W_aproj, b_aproj, x_group, W_gproj, b_gproj):
    Na = x_atom.shape[0]
    Gm = x_group.shape[0]
    out = pl.pallas_call(
        _tc_pre_body,
        out_shape=(
            jax.ShapeDtypeStruct((Na, 80), jnp.float32),
            jax.ShapeDtypeStruct((Gm, 80), jnp.float32),
            jax.ShapeDtypeStruct((Gm,), jnp.int32),
        ),
    )(x_atom, W_aproj, b_aproj.reshape(1, -1), x_group, W_gproj,
      b_gproj.reshape(1, -1))
    return out


def kernel(x_atom, atom_idx, x_group, group_idx, edge_index_group, cond_atom,
           W_gproj, b_gproj, W_aproj, b_aproj, Wih, Whh, bih, bhh,
           W_merge, b_merge, Wg1, bg1, Wg2, bg2, Wb1, bb1, Wb2, bb2):
    Gm = x_group.shape[0]
    Ninc = atom_idx.shape[0]

    xa_proj, xg0, type_ids = _tc_pre(x_atom, W_aproj, b_aproj,
                                     x_group, W_gproj, b_gproj)

    xs, cs, cn = _sc_seg_sum(atom_idx, group_idx, xa_proj, cond_atom, Gm)

    q2, cond_g, gamma, beta, cntc = _tc_mid(
        xs, cs, cn, Wih, Whh, bih, bhh, Wg1, bg1, Wg2, bg2,
        Wb1, bb1, Wb2, bb2, Gm)

    r2p, denp = _sc_attn(atom_idx, group_idx, xa_proj, q2, Gm)

    xg = _tc_fin(xg0, q2, r2p, denp, cntc, gamma, beta, cond_g,
                 W_merge, b_merge)
    return (x_atom, xg, type_ids)


# speedup vs baseline: 1.1871x; 1.1871x over previous
import functools

import jax
import jax.numpy as jnp
from jax import lax
from jax.experimental import pallas as pl
from jax.experimental.pallas import tpu as pltpu
from jax.experimental.pallas import tpu_sc as plsc

_NC, _NS = 2, 16
_NW = _NC * _NS
_BLK = 128


def _sc_seg_sum_body(idx2_hbm, xa_hbm, cond_hbm, z80, z64, z1,
                     xsum_out, csum_out, cnt_out,
                     ib, xrows, crows, ones,
                     xsum_sh, csum_sh, cnt_sh, sem_g, sem_s):
    c = lax.axis_index("c")
    s = lax.axis_index("s")
    wid = s * _NC + c
    Gp = cnt_sh.shape[0]
    nblk_tot = idx2_hbm.shape[0]

    for i in range(_BLK // 16):
        ones[pl.ds(16 * i, 16)] = jnp.ones((16,), jnp.float32)

    rows_per_s = Gp // _NS
    r0 = s * rows_per_s
    pltpu.sync_copy(z80.at[pl.ds(r0, rows_per_s)], xsum_sh.at[pl.ds(r0, rows_per_s)])
    pltpu.sync_copy(z64.at[pl.ds(r0, rows_per_s)], csum_sh.at[pl.ds(r0, rows_per_s)])
    pltpu.sync_copy(z1.at[pl.ds(r0, rows_per_s)], cnt_sh.at[pl.ds(r0, rows_per_s)])
    plsc.subcore_barrier()

    nblk_w = (nblk_tot - wid + _NW - 1) // _NW

    def body(i, carry):
        b = wid + i * _NW
        pltpu.sync_copy(idx2_hbm.at[b], ib)
        gx = pltpu.async_copy(xa_hbm.at[ib.at[0]], xrows, sem_g)
        gc = pltpu.async_copy(cond_hbm.at[ib.at[0]], crows, sem_g)
        gx.wait()
        gc.wait()
        s1 = pltpu.async_copy(xrows, xsum_sh.at[ib.at[1]], sem_s, add=True)
        s2 = pltpu.async_copy(crows, csum_sh.at[ib.at[1]], sem_s, add=True)
        s3 = pltpu.async_copy(ones, cnt_sh.at[ib.at[1]], sem_s, add=True)
        s1.wait()
        s2.wait()
        s3.wait()
        return carry

    lax.fori_loop(0, nblk_w, body, 0)
    plsc.subcore_barrier()

    pltpu.sync_copy(xsum_sh.at[pl.ds(r0, rows_per_s)],
                    xsum_out.at[c, pl.ds(r0, rows_per_s)])
    pltpu.sync_copy(csum_sh.at[pl.ds(r0, rows_per_s)],
                    csum_out.at[c, pl.ds(r0, rows_per_s)])
    pltpu.sync_copy(cnt_sh.at[pl.ds(r0, rows_per_s)],
                    cnt_out.at[c, pl.ds(r0, rows_per_s)])


def _mm(x, W):
    return lax.dot_general(x, W, (((1,), (1,)), ((), ())),
                           preferred_element_type=jnp.float32)


def _sigmoid(x):
    return 1.0 / (1.0 + jnp.exp(-x))


def _tc_mid_body(xs_ref, cs_ref, cn_ref, Wih_ref, Whh_ref, bih_ref, bhh_ref,
                 Wg1_ref, bg1_ref, Wg2_ref, bg2_ref,
                 Wb1_ref, bb1_ref, Wb2_ref, bb2_ref,
                 q2_ref, cond_ref, gamma_ref, beta_ref, cnt_ref):
    Gm = q2_ref.shape[0]
    D = q2_ref.shape[1]
    cnt = cn_ref[0, :Gm] + cn_ref[1, :Gm]
    xsum = xs_ref[0, :Gm] + xs_ref[1, :Gm]
    csum = cs_ref[0, :Gm] + cs_ref[1, :Gm]
    cntc = cnt[:, None]
    r1 = xsum / (cntc + 1e-16)
    cond_g = csum / jnp.maximum(cntc, 1.0)

    bihv = bih_ref[...]
    bhhv = bhh_ref[...]
    gates1 = bihv + bhhv
    i1 = _sigmoid(gates1[:, 0:D])
    g1 = jnp.tanh(gates1[:, 2 * D:3 * D])
    o1 = _sigmoid(gates1[:, 3 * D:4 * D])
    c1 = i1 * g1
    h1 = o1 * jnp.tanh(c1)
    h1b = jnp.broadcast_to(h1, (Gm, D))
    q_star1 = jnp.concatenate([h1b, r1], axis=1)

    gates = _mm(q_star1, Wih_ref[...]) + bihv + _mm(h1b, Whh_ref[...]) + bhhv
    i2 = _sigmoid(gates[:, 0:D])
    f2 = _sigmoid(gates[:, D:2 * D])
    g2 = jnp.tanh(gates[:, 2 * D:3 * D])
    o2 = _sigmoid(gates[:, 3 * D:4 * D])
    c2 = f2 * jnp.broadcast_to(c1, (Gm, D)) + i2 * g2
    q2_ref[...] = o2 * jnp.tanh(c2)

    cond_ref[...] = cond_g
    gamma_ref[...] = _mm(jnp.maximum(_mm(cond_g, Wg1_ref[...]) + bg1_ref[...],
                                     0.0), Wg2_ref[...]) + bg2_ref[...]
    beta_ref[...] = _mm(jnp.maximum(_mm(cond_g, Wb1_ref[...]) + bb1_ref[...],
                                    0.0), Wb2_ref[...]) + bb2_ref[...]
    cnt_ref[...] = cntc


def _tc_mid(xs, cs, cn, Wih, Whh, bih, bhh, Wg1, bg1, Wg2, bg2,
            Wb1, bb1, Wb2, bb2, Gm):
    D = 80
    Dc = cs.shape[2]
    return pl.pallas_call(
        _tc_mid_body,
        out_shape=(
            jax.ShapeDtypeStruct((Gm, D), jnp.float32),
            jax.ShapeDtypeStruct((Gm, Dc), jnp.float32),
            jax.ShapeDtypeStruct((Gm, 199), jnp.float32),
            jax.ShapeDtypeStruct((Gm, 199), jnp.float32),
            jax.ShapeDtypeStruct((Gm, 1), jnp.float32),
        ),
    )(xs, cs, cn, Wih, Whh, bih.reshape(1, -1), bhh.reshape(1, -1),
      Wg1, bg1.reshape(1, -1), Wg2, bg2.reshape(1, -1),
      Wb1, bb1.reshape(1, -1), Wb2, bb2.reshape(1, -1))


def _tc_fin_body(xg0_ref, q2_ref, r2_ref, den_ref, cnt_ref, gamma_ref,
                 beta_ref, cond_ref, Wm_ref, bm_ref, out_ref):
    Gm = xg0_ref.shape[0]
    r2 = ((r2_ref[0, :Gm] + r2_ref[1, :Gm])
          / (den_ref[0, :Gm] + den_ref[1, :Gm] + 1e-16)[:, None])
    present = cnt_ref[...] > 0.0
    q_star = jnp.where(present, jnp.concatenate([q2_ref[...], r2], axis=1), 0.0)
    xg = _mm(jnp.concatenate([xg0_ref[...], q_star], axis=1), Wm_ref[...]) \
        + bm_ref[...]
    xg = gamma_ref[...] * xg + beta_ref[...]
    out_ref[...] = jnp.concatenate([xg, cond_ref[...]], axis=1)


def _tc_fin(xg0, q2, r2p, denp, cnt, gamma, beta, cond_g, W_merge, b_merge):
    Gm = xg0.shape[0]
    return pl.pallas_call(
        _tc_fin_body,
        out_shape=jax.ShapeDtypeStruct((Gm, 263), jnp.float32),
    )(xg0, q2, r2p, denp, cnt, gamma, beta, cond_g, W_merge,
      b_merge.reshape(1, -1))


def _sc_attn_body(idx2_hbm, xa_hbm, q2_hbm, z80, z1,
                  r2_out, den_out,
                  ib, xrows, qrows, srows, wblk, eloc,
                  r2_sh, den_sh, sem_g, sem_s):
    c = lax.axis_index("c")
    s = lax.axis_index("s")
    wid = s * _NC + c
    Gp = den_sh.shape[0]
    nblk_tot = idx2_hbm.shape[0]
    D = xrows.shape[1]
    nch = D // 16

    rows_per_s = Gp // _NS
    r0 = s * rows_per_s
    pltpu.sync_copy(z80.at[pl.ds(r0, rows_per_s)], r2_sh.at[pl.ds(r0, rows_per_s)])
    pltpu.sync_copy(z1.at[pl.ds(r0, rows_per_s)], den_sh.at[pl.ds(r0, rows_per_s)])
    plsc.subcore_barrier()

    m15 = lax.iota(jnp.int32, 16) == 15
    nblk_w = (nblk_tot - wid + _NW - 1) // _NW

    def block_body(bi, carry):
        b = wid + bi * _NW
        pltpu.sync_copy(idx2_hbm.at[b], ib)
        gx = pltpu.async_copy(xa_hbm.at[ib.at[0]], xrows, sem_g)
        gq = pltpu.async_copy(q2_hbm.at[ib.at[1]], qrows, sem_g)
        gx.wait()
        gq.wait()

        def tile_body(j, carry2):
            for i in range(16):
                row = j * 16 + i
                p = xrows[row, pl.ds(0, 16)] * qrows[row, pl.ds(0, 16)]
                for ch in range(1, nch):
                    p = p + (xrows[row, pl.ds(16 * ch, 16)]
                             * qrows[row, pl.ds(16 * ch, 16)])
                plsc.store_compressed(eloc.at[pl.ds(i, 16)], plsc.cumsum(p),
                                      mask=m15)
            wv = jnp.exp(eloc[pl.ds(0, 16)])
            wblk[pl.ds(j * 16, 16)] = wv
            for i in range(16):
                row = j * 16 + i
                ws = wv[i]
                for ch in range(nch):
                    srows[row, pl.ds(16 * ch, 16)] = (
                        xrows[row, pl.ds(16 * ch, 16)] * ws)
            return carry2

        lax.fori_loop(0, _BLK // 16, tile_body, 0)
        s1 = pltpu.async_copy(srows, r2_sh.at[ib.at[1]], sem_s, add=True)
        s2 = pltpu.async_copy(wblk, den_sh.at[ib.at[1]], sem_s, add=True)
        s1.wait()
        s2.wait()
        return carry

    lax.fori_loop(0, nblk_w, block_body, 0)
    plsc.subcore_barrier()

    pltpu.sync_copy(r2_sh.at[pl.ds(r0, rows_per_s)],
                    r2_out.at[c, pl.ds(r0, rows_per_s)])
    pltpu.sync_copy(den_sh.at[pl.ds(r0, rows_per_s)],
                    den_out.at[c, pl.ds(r0, rows_per_s)])


def _sc_attn(idx2, xa_proj, q2, Gm):
    Dx = xa_proj.shape[1]
    Gp = ((Gm + _NS * 8 - 1) // (_NS * 8)) * (_NS * 8)
    z80 = jnp.zeros((Gp, Dx), jnp.float32)
    z1 = jnp.zeros((Gp,), jnp.float32)
    f = pl.kernel(
        _sc_attn_body,
        out_type=(
            jax.ShapeDtypeStruct((2, Gp, Dx), jnp.float32),
            jax.ShapeDtypeStruct((2, Gp), jnp.float32),
        ),
        mesh=plsc.VectorSubcoreMesh(core_axis_name="c", subcore_axis_name="s"),
        compiler_params=pltpu.CompilerParams(use_tc_tiling_on_sc=False,
                                             needs_layout_passes=False),
        scratch_types=[
            pltpu.VMEM((2, _BLK), jnp.int32),
            pltpu.VMEM((_BLK, Dx), jnp.float32),
            pltpu.VMEM((_BLK, Dx), jnp.float32),
            pltpu.VMEM((_BLK, Dx), jnp.float32),
            pltpu.VMEM((_BLK,), jnp.float32),
            pltpu.VMEM((32,), jnp.float32),
            pltpu.VMEM_SHARED((Gp, Dx), jnp.float32),
            pltpu.VMEM_SHARED((Gp,), jnp.float32),
            pltpu.SemaphoreType.DMA,
            pltpu.SemaphoreType.DMA,
        ],
    )
    return f(idx2, xa_proj, q2, z80, z1)


def _sc_seg_sum(idx2, xa_proj, cond_atom, Gm):
    Dx = xa_proj.shape[1]
    Dc = cond_atom.shape[1]
    Gp = ((Gm + _NS * 8 - 1) // (_NS * 8)) * (_NS * 8)
    z80 = jnp.zeros((Gp, Dx), jnp.float32)
    z64 = jnp.zeros((Gp, Dc), jnp.float32)
    z1 = jnp.zeros((Gp,), jnp.float32)
    f = pl.kernel(
        _sc_seg_sum_body,
        out_type=(
            jax.ShapeDtypeStruct((2, Gp, Dx), jnp.float32),
            jax.ShapeDtypeStruct((2, Gp, Dc), jnp.float32),
            jax.ShapeDtypeStruct((2, Gp), jnp.float32),
        ),
        mesh=plsc.VectorSubcoreMesh(core_axis_name="c", subcore_axis_name="s"),
        compiler_params=pltpu.CompilerParams(use_tc_tiling_on_sc=False),
        scratch_types=[
            pltpu.VMEM((2, _BLK), jnp.int32),
            pltpu.VMEM((_BLK, Dx), jnp.float32),
            pltpu.VMEM((_BLK, Dc), jnp.float32),
            pltpu.VMEM((_BLK,), jnp.float32),
            pltpu.VMEM_SHARED((Gp, Dx), jnp.float32),
            pltpu.VMEM_SHARED((Gp, Dc), jnp.float32),
            pltpu.VMEM_SHARED((Gp,), jnp.float32),
            pltpu.SemaphoreType.DMA,
            pltpu.SemaphoreType.DMA,
        ],
    )
    return f(idx2, xa_proj, cond_atom, z80, z64, z1)


def _tc_pre_body(x_atom_ref, W_a_ref, b_a_ref, x_group_ref, W_g_ref, b_g_ref,
                 xa_ref, xg0_ref, tid_ref):
    xa_ref[...] = lax.dot_general(
        x_atom_ref[...], W_a_ref[...], (((1,), (1,)), ((), ())),
        preferred_element_type=jnp.float32) + b_a_ref[...]
    Xg = x_group_ref[...][:, 0:40]
    xg0_ref[...] = lax.dot_general(
        Xg, W_g_ref[...], (((1,), (1,)), ((), ())),
        preferred_element_type=jnp.float32) + b_g_ref[...]
    m = jnp.max(Xg, axis=1, keepdims=True)
    cols = lax.broadcasted_iota(jnp.int32, Xg.shape, 1)
    tid = jnp.min(jnp.where(Xg == m, cols, jnp.int32(2**30)), axis=1)
    tid_ref[...] = tid.astype(jnp.int32)


def _tc_pre(x_atom, W_aproj, b_aproj, x_group, W_gproj, b_gproj):
    Na = x_atom.shape[0]
    Gm = x_group.shape[0]
    out = pl.pallas_call(
        _tc_pre_body,
        out_shape=(
            jax.ShapeDtypeStruct((Na, 80), jnp.float32),
            jax.ShapeDtypeStruct((Gm, 80), jnp.float32),
            jax.ShapeDtypeStruct((Gm,), jnp.int32),
        ),
    )(x_atom, W_aproj, b_aproj.reshape(1, -1), x_group, W_gproj,
      b_gproj.reshape(1, -1))
    return out


def kernel(x_atom, atom_idx, x_group, group_idx, edge_index_group, cond_atom,
           W_gproj, b_gproj, W_aproj, b_aproj, Wih, Whh, bih, bhh,
           W_merge, b_merge, Wg1, bg1, Wg2, bg2, Wb1, bb1, Wb2, bb2):
    Gm = x_group.shape[0]
    Ninc = atom_idx.shape[0]

    xa_proj, xg0, type_ids = _tc_pre(x_atom, W_aproj, b_aproj,
                                     x_group, W_gproj, b_gproj)

    idx2 = jnp.stack([atom_idx.reshape(-1, _BLK),
                      group_idx.reshape(-1, _BLK)], axis=1)
    xs, cs, cn = _sc_seg_sum(idx2, xa_proj, cond_atom, Gm)

    q2, cond_g, gamma, beta, cntc = _tc_mid(
        xs, cs, cn, Wih, Whh, bih, bhh, Wg1, bg1, Wg2, bg2,
        Wb1, bb1, Wb2, bb2, Gm)

    r2p, denp = _sc_attn(idx2, xa_proj, q2, Gm)

    xg = _tc_fin(xg0, q2, r2p, denp, cntc, gamma, beta, cond_g,
                 W_merge, b_merge)
    return (x_atom, xg, type_ids)


# speedup vs baseline: 1.4845x; 1.2505x over previous
import functools

import jax
import jax.numpy as jnp
from jax import lax
from jax.experimental import pallas as pl
from jax.experimental.pallas import tpu as pltpu
from jax.experimental.pallas import tpu_sc as plsc

_NC, _NS = 2, 16
_NW = _NC * _NS
_BLK = 128


def _sc_seg_sum_body(idx2_hbm, xa_hbm, cond_hbm, z80, z64, z1,
                     xsum_out, csum_out, cnt_out,
                     ib, xrows, crows, ones,
                     xsum_sh, csum_sh, cnt_sh, sem_g, sem_s):
    c = lax.axis_index("c")
    s = lax.axis_index("s")
    wid = s * _NC + c
    Gp = cnt_sh.shape[0]
    nblk_tot = idx2_hbm.shape[0]

    for i in range(_BLK // 16):
        ones[pl.ds(16 * i, 16)] = jnp.ones((16,), jnp.float32)

    rows_per_s = Gp // _NS
    r0 = s * rows_per_s
    pltpu.sync_copy(z80.at[pl.ds(r0, rows_per_s)], xsum_sh.at[pl.ds(r0, rows_per_s)])
    pltpu.sync_copy(z64.at[pl.ds(r0, rows_per_s)], csum_sh.at[pl.ds(r0, rows_per_s)])
    pltpu.sync_copy(z1.at[pl.ds(r0, rows_per_s)], cnt_sh.at[pl.ds(r0, rows_per_s)])
    plsc.subcore_barrier()

    nblk_w = (nblk_tot - wid + _NW - 1) // _NW

    def body(i, carry):
        b = wid + i * _NW
        pltpu.sync_copy(idx2_hbm.at[b], ib)
        gx = pltpu.async_copy(xa_hbm.at[ib.at[0]], xrows, sem_g)
        gc = pltpu.async_copy(cond_hbm.at[ib.at[0]], crows, sem_g)
        gx.wait()
        gc.wait()
        s1 = pltpu.async_copy(xrows, xsum_sh.at[ib.at[1]], sem_s, add=True)
        s2 = pltpu.async_copy(crows, csum_sh.at[ib.at[1]], sem_s, add=True)
        s3 = pltpu.async_copy(ones, cnt_sh.at[ib.at[1]], sem_s, add=True)
        s1.wait()
        s2.wait()
        s3.wait()
        return carry

    lax.fori_loop(0, nblk_w, body, 0)
    plsc.subcore_barrier()

    pltpu.sync_copy(xsum_sh.at[pl.ds(r0, rows_per_s)],
                    xsum_out.at[c, pl.ds(r0, rows_per_s)])
    pltpu.sync_copy(csum_sh.at[pl.ds(r0, rows_per_s)],
                    csum_out.at[c, pl.ds(r0, rows_per_s)])
    pltpu.sync_copy(cnt_sh.at[pl.ds(r0, rows_per_s)],
                    cnt_out.at[c, pl.ds(r0, rows_per_s)])


def _mm(x, W):
    return lax.dot_general(x, W, (((1,), (1,)), ((), ())),
                           preferred_element_type=jnp.float32)


def _sigmoid(x):
    return 1.0 / (1.0 + jnp.exp(-x))


def _tc_mid_body(xs_ref, cs_ref, cn_ref, Wih_ref, Whh_ref, bih_ref, bhh_ref,
                 Wg1_ref, bg1_ref, Wg2_ref, bg2_ref,
                 Wb1_ref, bb1_ref, Wb2_ref, bb2_ref,
                 q2_ref, cond_ref, gamma_ref, beta_ref, cnt_ref):
    Gm = q2_ref.shape[0]
    D = q2_ref.shape[1]
    cnt = cn_ref[0, :Gm] + cn_ref[1, :Gm]
    xsum = xs_ref[0, :Gm] + xs_ref[1, :Gm]
    csum = cs_ref[0, :Gm] + cs_ref[1, :Gm]
    cntc = cnt[:, None]
    r1 = xsum / (cntc + 1e-16)
    cond_g = csum / jnp.maximum(cntc, 1.0)

    bihv = bih_ref[...]
    bhhv = bhh_ref[...]
    gates1 = bihv + bhhv
    i1 = _sigmoid(gates1[:, 0:D])
    g1 = jnp.tanh(gates1[:, 2 * D:3 * D])
    o1 = _sigmoid(gates1[:, 3 * D:4 * D])
    c1 = i1 * g1
    h1 = o1 * jnp.tanh(c1)
    h1b = jnp.broadcast_to(h1, (Gm, D))
    q_star1 = jnp.concatenate([h1b, r1], axis=1)

    gates = _mm(q_star1, Wih_ref[...]) + bihv + _mm(h1b, Whh_ref[...]) + bhhv
    i2 = _sigmoid(gates[:, 0:D])
    f2 = _sigmoid(gates[:, D:2 * D])
    g2 = jnp.tanh(gates[:, 2 * D:3 * D])
    o2 = _sigmoid(gates[:, 3 * D:4 * D])
    c2 = f2 * jnp.broadcast_to(c1, (Gm, D)) + i2 * g2
    q2_ref[...] = o2 * jnp.tanh(c2)

    cond_ref[...] = cond_g
    gamma_ref[...] = _mm(jnp.maximum(_mm(cond_g, Wg1_ref[...]) + bg1_ref[...],
                                     0.0), Wg2_ref[...]) + bg2_ref[...]
    beta_ref[...] = _mm(jnp.maximum(_mm(cond_g, Wb1_ref[...]) + bb1_ref[...],
                                    0.0), Wb2_ref[...]) + bb2_ref[...]
    cnt_ref[...] = cntc


def _tc_mid(xs, cs, cn, Wih, Whh, bih, bhh, Wg1, bg1, Wg2, bg2,
            Wb1, bb1, Wb2, bb2, Gm):
    D = 80
    Dc = cs.shape[2]
    return pl.pallas_call(
        _tc_mid_body,
        out_shape=(
            jax.ShapeDtypeStruct((Gm, D), jnp.float32),
            jax.ShapeDtypeStruct((Gm, Dc), jnp.float32),
            jax.ShapeDtypeStruct((Gm, 199), jnp.float32),
            jax.ShapeDtypeStruct((Gm, 199), jnp.float32),
            jax.ShapeDtypeStruct((Gm, 1), jnp.float32),
        ),
    )(xs, cs, cn, Wih, Whh, bih.reshape(1, -1), bhh.reshape(1, -1),
      Wg1, bg1.reshape(1, -1), Wg2, bg2.reshape(1, -1),
      Wb1, bb1.reshape(1, -1), Wb2, bb2.reshape(1, -1))


def _tc_fin_body(xg0_ref, q2_ref, r2_ref, den_ref, cnt_ref, gamma_ref,
                 beta_ref, cond_ref, Wm_ref, bm_ref, out_ref):
    Gm = xg0_ref.shape[0]
    r2 = ((r2_ref[0, :Gm] + r2_ref[1, :Gm])
          / (den_ref[0, :Gm] + den_ref[1, :Gm] + 1e-16)[:, None])
    present = cnt_ref[...] > 0.0
    q_star = jnp.where(present, jnp.concatenate([q2_ref[...], r2], axis=1), 0.0)
    xg = _mm(jnp.concatenate([xg0_ref[...], q_star], axis=1), Wm_ref[...]) \
        + bm_ref[...]
    xg = gamma_ref[...] * xg + beta_ref[...]
    out_ref[...] = jnp.concatenate([xg, cond_ref[...]], axis=1)


def _tc_fin(xg0, q2, r2p, denp, cnt, gamma, beta, cond_g, W_merge, b_merge):
    Gm = xg0.shape[0]
    return pl.pallas_call(
        _tc_fin_body,
        out_shape=jax.ShapeDtypeStruct((Gm, 263), jnp.float32),
    )(xg0, q2, r2p, denp, cnt, gamma, beta, cond_g, W_merge,
      b_merge.reshape(1, -1))


def _sc_attn_body(idx2_hbm, xa_hbm, q2_hbm, z80, z1,
                  r2_out, den_out,
                  ib0, ib1, xr0, xr1, qr0, qr1, sr0, sr1, wb0, wb1,
                  gs0, gs1, eloc,
                  r2_sh, den_sh, sg0, sg1, ss0, ss1):
    c = lax.axis_index("c")
    s = lax.axis_index("s")
    wid = s * _NC + c
    Gp = den_sh.shape[0]
    nblk_tot = idx2_hbm.shape[0]
    D = xr0.shape[1]
    nch = D // 16
    ibs, xrs, qrs = (ib0, ib1), (xr0, xr1), (qr0, qr1)
    srs, wbs, gss = (sr0, sr1), (wb0, wb1), (gs0, gs1)
    sgs, sss = (sg0, sg1), (ss0, ss1)

    rows_per_s = Gp // _NS
    r0 = s * rows_per_s
    pltpu.sync_copy(z80.at[pl.ds(r0, rows_per_s)], r2_sh.at[pl.ds(r0, rows_per_s)])
    pltpu.sync_copy(z1.at[pl.ds(r0, rows_per_s)], den_sh.at[pl.ds(r0, rows_per_s)])
    plsc.subcore_barrier()

    m15 = lax.iota(jnp.int32, 16) == 15
    nblk_w = (nblk_tot - wid + _NW - 1) // _NW

    def start_loads(i, k):
        b = wid + i * _NW
        pltpu.sync_copy(idx2_hbm.at[b], ibs[k])
        pltpu.async_copy(xa_hbm.at[ibs[k].at[0]], xrs[k], sgs[k])
        pltpu.async_copy(q2_hbm.at[ibs[k].at[1]], qrs[k], sgs[k])

    def drain_scatters(k):
        pltpu.make_async_copy(srs[k], r2_sh.at[gss[k]], sss[k]).wait()
        pltpu.make_async_copy(wbs[k], den_sh.at[gss[k]], sss[k]).wait()

    start_loads(0, 0)

    def pair(ii, carry):
        for k in (0, 1):
            i = ii * 2 + k

            @pl.when(i < nblk_w)
            def _do(i=i, k=k):
                @pl.when(i >= 2)
                def _dr():
                    drain_scatters(k)

                pltpu.make_async_copy(xa_hbm.at[ibs[k].at[0]], xrs[k],
                                      sgs[k]).wait()
                pltpu.make_async_copy(q2_hbm.at[ibs[k].at[1]], qrs[k],
                                      sgs[k]).wait()

                @pl.when(i + 1 < nblk_w)
                def _pf():
                    start_loads(i + 1, k ^ 1)

                for t in range(_BLK // 16):
                    gss[k][pl.ds(16 * t, 16)] = ibs[k][1, pl.ds(16 * t, 16)]

                xrows, qrows, srows, wblk = xrs[k], qrs[k], srs[k], wbs[k]

                def tile_body(j, carry2):
                    for i2 in range(16):
                        row = j * 16 + i2
                        p = xrows[row, pl.ds(0, 16)] * qrows[row, pl.ds(0, 16)]
                        for ch in range(1, nch):
                            p = p + (xrows[row, pl.ds(16 * ch, 16)]
                                     * qrows[row, pl.ds(16 * ch, 16)])
                        plsc.store_compressed(eloc.at[pl.ds(i2, 16)],
                                              plsc.cumsum(p), mask=m15)
                    wv = jnp.exp(eloc[pl.ds(0, 16)])
                    wblk[pl.ds(j * 16, 16)] = wv
                    for i2 in range(16):
                        row = j * 16 + i2
                        ws = wv[i2]
                        for ch in range(nch):
                            srows[row, pl.ds(16 * ch, 16)] = (
                                xrows[row, pl.ds(16 * ch, 16)] * ws)
                    return carry2

                lax.fori_loop(0, _BLK // 16, tile_body, 0)
                pltpu.async_copy(srows, r2_sh.at[gss[k]], sss[k], add=True)
                pltpu.async_copy(wblk, den_sh.at[gss[k]], sss[k], add=True)
        return carry

    lax.fori_loop(0, (nblk_w + 1) // 2, pair, 0)
    for k in (0, 1):
        @pl.when(nblk_w > k)
        def _fin(k=k):
            drain_scatters(k)
    plsc.subcore_barrier()

    pltpu.sync_copy(r2_sh.at[pl.ds(r0, rows_per_s)],
                    r2_out.at[c, pl.ds(r0, rows_per_s)])
    pltpu.sync_copy(den_sh.at[pl.ds(r0, rows_per_s)],
                    den_out.at[c, pl.ds(r0, rows_per_s)])


def _sc_attn(idx2, xa_proj, q2, Gm):
    Dx = xa_proj.shape[1]
    Gp = ((Gm + _NS * 8 - 1) // (_NS * 8)) * (_NS * 8)
    z80 = jnp.zeros((Gp, Dx), jnp.float32)
    z1 = jnp.zeros((Gp,), jnp.float32)
    f = pl.kernel(
        _sc_attn_body,
        out_type=(
            jax.ShapeDtypeStruct((2, Gp, Dx), jnp.float32),
            jax.ShapeDtypeStruct((2, Gp), jnp.float32),
        ),
        mesh=plsc.VectorSubcoreMesh(core_axis_name="c", subcore_axis_name="s"),
        compiler_params=pltpu.CompilerParams(use_tc_tiling_on_sc=False,
                                             needs_layout_passes=False),
        scratch_types=[
            pltpu.VMEM((2, _BLK), jnp.int32),
            pltpu.VMEM((2, _BLK), jnp.int32),
            pltpu.VMEM((_BLK, Dx), jnp.float32),
            pltpu.VMEM((_BLK, Dx), jnp.float32),
            pltpu.VMEM((_BLK, Dx), jnp.float32),
            pltpu.VMEM((_BLK, Dx), jnp.float32),
            pltpu.VMEM((_BLK, Dx), jnp.float32),
            pltpu.VMEM((_BLK, Dx), jnp.float32),
            pltpu.VMEM((_BLK,), jnp.float32),
            pltpu.VMEM((_BLK,), jnp.float32),
            pltpu.VMEM((_BLK,), jnp.int32),
            pltpu.VMEM((_BLK,), jnp.int32),
            pltpu.VMEM((32,), jnp.float32),
            pltpu.VMEM_SHARED((Gp, Dx), jnp.float32),
            pltpu.VMEM_SHARED((Gp,), jnp.float32),
            pltpu.SemaphoreType.DMA,
            pltpu.SemaphoreType.DMA,
            pltpu.SemaphoreType.DMA,
            pltpu.SemaphoreType.DMA,
        ],
    )
    return f(idx2, xa_proj, q2, z80, z1)


def _sc_seg_sum(idx2, xa_proj, cond_atom, Gm):
    Dx = xa_proj.shape[1]
    Dc = cond_atom.shape[1]
    Gp = ((Gm + _NS * 8 - 1) // (_NS * 8)) * (_NS * 8)
    z80 = jnp.zeros((Gp, Dx), jnp.float32)
    z64 = jnp.zeros((Gp, Dc), jnp.float32)
    z1 = jnp.zeros((Gp,), jnp.float32)
    f = pl.kernel(
        _sc_seg_sum_body,
        out_type=(
            jax.ShapeDtypeStruct((2, Gp, Dx), jnp.float32),
            jax.ShapeDtypeStruct((2, Gp, Dc), jnp.float32),
            jax.ShapeDtypeStruct((2, Gp), jnp.float32),
        ),
        mesh=plsc.VectorSubcoreMesh(core_axis_name="c", subcore_axis_name="s"),
        compiler_params=pltpu.CompilerParams(use_tc_tiling_on_sc=False),
        scratch_types=[
            pltpu.VMEM((2, _BLK), jnp.int32),
            pltpu.VMEM((_BLK, Dx), jnp.float32),
            pltpu.VMEM((_BLK, Dc), jnp.float32),
            pltpu.VMEM((_BLK,), jnp.float32),
            pltpu.VMEM_SHARED((Gp, Dx), jnp.float32),
            pltpu.VMEM_SHARED((Gp, Dc), jnp.float32),
            pltpu.VMEM_SHARED((Gp,), jnp.float32),
            pltpu.SemaphoreType.DMA,
            pltpu.SemaphoreType.DMA,
        ],
    )
    return f(idx2, xa_proj, cond_atom, z80, z64, z1)


def _tc_pre_body(x_atom_ref, W_a_ref, b_a_ref, x_group_ref, W_g_ref, b_g_ref,
                 xa_ref, xg0_ref, tid_ref):
    xa_ref[...] = lax.dot_general(
        x_atom_ref[...], W_a_ref[...], (((1,), (1,)), ((), ())),
        preferred_element_type=jnp.float32) + b_a_ref[...]
    Xg = x_group_ref[...][:, 0:40]
    xg0_ref[...] = lax.dot_general(
        Xg, W_g_ref[...], (((1,), (1,)), ((), ())),
        preferred_element_type=jnp.float32) + b_g_ref[...]
    m = jnp.max(Xg, axis=1, keepdims=True)
    cols = lax.broadcasted_iota(jnp.int32, Xg.shape, 1)
    tid = jnp.min(jnp.where(Xg == m, cols, jnp.int32(2**30)), axis=1)
    tid_ref[...] = tid.astype(jnp.int32)


def _tc_pre(x_atom, W_aproj, b_aproj, x_group, W_gproj, b_gproj):
    Na = x_atom.shape[0]
    Gm = x_group.shape[0]
    out = pl.pallas_call(
        _tc_pre_body,
        out_shape=(
            jax.ShapeDtypeStruct((Na, 80), jnp.float32),
            jax.ShapeDtypeStruct((Gm, 80), jnp.float32),
            jax.ShapeDtypeStruct((Gm,), jnp.int32),
        ),
    )(x_atom, W_aproj, b_aproj.reshape(1, -1), x_group, W_gproj,
      b_gproj.reshape(1, -1))
    return out


def kernel(x_atom, atom_idx, x_group, group_idx, edge_index_group, cond_atom,
           W_gproj, b_gproj, W_aproj, b_aproj, Wih, Whh, bih, bhh,
           W_merge, b_merge, Wg1, bg1, Wg2, bg2, Wb1, bb1, Wb2, bb2):
    Gm = x_group.shape[0]
    Ninc = atom_idx.shape[0]

    xa_proj, xg0, type_ids = _tc_pre(x_atom, W_aproj, b_aproj,
                                     x_group, W_gproj, b_gproj)

    idx2 = jnp.stack([atom_idx.reshape(-1, _BLK),
                      group_idx.reshape(-1, _BLK)], axis=1)
    xs, cs, cn = _sc_seg_sum(idx2, xa_proj, cond_atom, Gm)

    q2, cond_g, gamma, beta, cntc = _tc_mid(
        xs, cs, cn, Wih, Whh, bih, bhh, Wg1, bg1, Wg2, bg2,
        Wb1, bb1, Wb2, bb2, Gm)

    r2p, denp = _sc_attn(idx2, xa_proj, q2, Gm)

    xg = _tc_fin(xg0, q2, r2p, denp, cntc, gamma, beta, cond_g,
                 W_merge, b_merge)
    return (x_atom, xg, type_ids)


# speedup vs baseline: 1.5867x; 1.0689x over previous
import functools

import jax
import jax.numpy as jnp
from jax import lax
from jax.experimental import pallas as pl
from jax.experimental.pallas import tpu as pltpu
from jax.experimental.pallas import tpu_sc as plsc

_NC, _NS = 2, 16
_NW = _NC * _NS
_BLK = 128


def _sc_seg_sum_body(idx2_hbm, xa_hbm, cond_hbm, z80, z64, z1,
                     xsum_out, csum_out, cnt_out,
                     ib0, ib1, xr0, xr1, cr0, cr1, gs0, gs1, ones,
                     xsum_sh, csum_sh, cnt_sh, sg0, sg1, ss0, ss1):
    c = lax.axis_index("c")
    s = lax.axis_index("s")
    wid = s * _NC + c
    Gp = cnt_sh.shape[0]
    nblk_tot = idx2_hbm.shape[0]

    for i in range(_BLK // 16):
        ones[pl.ds(16 * i, 16)] = jnp.ones((16,), jnp.float32)

    rows_per_s = Gp // _NS
    r0 = s * rows_per_s
    pltpu.sync_copy(z80.at[pl.ds(r0, rows_per_s)], xsum_sh.at[pl.ds(r0, rows_per_s)])
    pltpu.sync_copy(z64.at[pl.ds(r0, rows_per_s)], csum_sh.at[pl.ds(r0, rows_per_s)])
    pltpu.sync_copy(z1.at[pl.ds(r0, rows_per_s)], cnt_sh.at[pl.ds(r0, rows_per_s)])
    plsc.subcore_barrier()

    nblk_w = (nblk_tot - wid + _NW - 1) // _NW
    ibs, xrs, crs = (ib0, ib1), (xr0, xr1), (cr0, cr1)
    gss, sgs, sss = (gs0, gs1), (sg0, sg1), (ss0, ss1)

    def start_loads(i, k):
        b = wid + i * _NW
        pltpu.sync_copy(idx2_hbm.at[b], ibs[k])
        pltpu.async_copy(xa_hbm.at[ibs[k].at[0]], xrs[k], sgs[k])
        pltpu.async_copy(cond_hbm.at[ibs[k].at[0]], crs[k], sgs[k])

    def drain_scatters(k):
        pltpu.make_async_copy(xrs[k], xsum_sh.at[gss[k]], sss[k]).wait()
        pltpu.make_async_copy(crs[k], csum_sh.at[gss[k]], sss[k]).wait()
        pltpu.make_async_copy(ones, cnt_sh.at[gss[k]], sss[k]).wait()

    start_loads(0, 0)

    def pair(ii, carry):
        for k in (0, 1):
            i = ii * 2 + k

            @pl.when(i < nblk_w)
            def _do(i=i, k=k):
                pltpu.make_async_copy(xa_hbm.at[ibs[k].at[0]], xrs[k],
                                      sgs[k]).wait()
                pltpu.make_async_copy(cond_hbm.at[ibs[k].at[0]], crs[k],
                                      sgs[k]).wait()
                for t in range(_BLK // 16):
                    gss[k][pl.ds(16 * t, 16)] = ibs[k][1, pl.ds(16 * t, 16)]

                @pl.when(i >= 1)
                def _dr():
                    drain_scatters(k ^ 1)

                @pl.when(i + 1 < nblk_w)
                def _pf():
                    start_loads(i + 1, k ^ 1)

                pltpu.async_copy(xrs[k], xsum_sh.at[gss[k]], sss[k], add=True)
                pltpu.async_copy(crs[k], csum_sh.at[gss[k]], sss[k], add=True)
                pltpu.async_copy(ones, cnt_sh.at[gss[k]], sss[k], add=True)
        return carry

    lax.fori_loop(0, (nblk_w + 1) // 2, pair, 0)
    for k in (0, 1):
        @pl.when((nblk_w - 1) % 2 == k)
        def _fin(k=k):
            drain_scatters(k)
    plsc.subcore_barrier()

    pltpu.sync_copy(xsum_sh.at[pl.ds(r0, rows_per_s)],
                    xsum_out.at[c, pl.ds(r0, rows_per_s)])
    pltpu.sync_copy(csum_sh.at[pl.ds(r0, rows_per_s)],
                    csum_out.at[c, pl.ds(r0, rows_per_s)])
    pltpu.sync_copy(cnt_sh.at[pl.ds(r0, rows_per_s)],
                    cnt_out.at[c, pl.ds(r0, rows_per_s)])


def _mm(x, W):
    return lax.dot_general(x, W, (((1,), (1,)), ((), ())),
                           preferred_element_type=jnp.float32)


def _sigmoid(x):
    return 1.0 / (1.0 + jnp.exp(-x))


def _tc_mid_body(xs_ref, cs_ref, cn_ref, Wih_ref, Whh_ref, bih_ref, bhh_ref,
                 Wg1_ref, bg1_ref, Wg2_ref, bg2_ref,
                 Wb1_ref, bb1_ref, Wb2_ref, bb2_ref,
                 q2_ref, cond_ref, gamma_ref, beta_ref, cnt_ref):
    Gm = q2_ref.shape[0]
    D = q2_ref.shape[1]
    cnt = cn_ref[0, :Gm] + cn_ref[1, :Gm]
    xsum = xs_ref[0, :Gm] + xs_ref[1, :Gm]
    csum = cs_ref[0, :Gm] + cs_ref[1, :Gm]
    cntc = cnt[:, None]
    r1 = xsum / (cntc + 1e-16)
    cond_g = csum / jnp.maximum(cntc, 1.0)

    bihv = bih_ref[...]
    bhhv = bhh_ref[...]
    gates1 = bihv + bhhv
    i1 = _sigmoid(gates1[:, 0:D])
    g1 = jnp.tanh(gates1[:, 2 * D:3 * D])
    o1 = _sigmoid(gates1[:, 3 * D:4 * D])
    c1 = i1 * g1
    h1 = o1 * jnp.tanh(c1)
    h1b = jnp.broadcast_to(h1, (Gm, D))
    q_star1 = jnp.concatenate([h1b, r1], axis=1)

    gates = _mm(q_star1, Wih_ref[...]) + bihv + _mm(h1b, Whh_ref[...]) + bhhv
    i2 = _sigmoid(gates[:, 0:D])
    f2 = _sigmoid(gates[:, D:2 * D])
    g2 = jnp.tanh(gates[:, 2 * D:3 * D])
    o2 = _sigmoid(gates[:, 3 * D:4 * D])
    c2 = f2 * jnp.broadcast_to(c1, (Gm, D)) + i2 * g2
    q2_ref[...] = o2 * jnp.tanh(c2)

    cond_ref[...] = cond_g
    gamma_ref[...] = _mm(jnp.maximum(_mm(cond_g, Wg1_ref[...]) + bg1_ref[...],
                                     0.0), Wg2_ref[...]) + bg2_ref[...]
    beta_ref[...] = _mm(jnp.maximum(_mm(cond_g, Wb1_ref[...]) + bb1_ref[...],
                                    0.0), Wb2_ref[...]) + bb2_ref[...]
    cnt_ref[...] = cntc


def _tc_mid(xs, cs, cn, Wih, Whh, bih, bhh, Wg1, bg1, Wg2, bg2,
            Wb1, bb1, Wb2, bb2, Gm):
    D = 80
    Dc = cs.shape[2]
    return pl.pallas_call(
        _tc_mid_body,
        out_shape=(
            jax.ShapeDtypeStruct((Gm, D), jnp.float32),
            jax.ShapeDtypeStruct((Gm, Dc), jnp.float32),
            jax.ShapeDtypeStruct((Gm, 199), jnp.float32),
            jax.ShapeDtypeStruct((Gm, 199), jnp.float32),
            jax.ShapeDtypeStruct((Gm, 1), jnp.float32),
        ),
    )(xs, cs, cn, Wih, Whh, bih.reshape(1, -1), bhh.reshape(1, -1),
      Wg1, bg1.reshape(1, -1), Wg2, bg2.reshape(1, -1),
      Wb1, bb1.reshape(1, -1), Wb2, bb2.reshape(1, -1))


def _tc_fin_body(xg0_ref, q2_ref, r2_ref, den_ref, cnt_ref, gamma_ref,
                 beta_ref, cond_ref, Wm_ref, bm_ref, out_ref):
    Gm = xg0_ref.shape[0]
    r2 = ((r2_ref[0, :Gm] + r2_ref[1, :Gm])
          / (den_ref[0, :Gm] + den_ref[1, :Gm] + 1e-16)[:, None])
    present = cnt_ref[...] > 0.0
    q_star = jnp.where(present, jnp.concatenate([q2_ref[...], r2], axis=1), 0.0)
    xg = _mm(jnp.concatenate([xg0_ref[...], q_star], axis=1), Wm_ref[...]) \
        + bm_ref[...]
    xg = gamma_ref[...] * xg + beta_ref[...]
    out_ref[...] = jnp.concatenate([xg, cond_ref[...]], axis=1)


def _tc_fin(xg0, q2, r2p, denp, cnt, gamma, beta, cond_g, W_merge, b_merge):
    Gm = xg0.shape[0]
    return pl.pallas_call(
        _tc_fin_body,
        out_shape=jax.ShapeDtypeStruct((Gm, 263), jnp.float32),
    )(xg0, q2, r2p, denp, cnt, gamma, beta, cond_g, W_merge,
      b_merge.reshape(1, -1))


def _sc_attn_body(idx2_hbm, xa_hbm, q2_hbm, z80, z1,
                  r2_out, den_out,
                  ib0, ib1, xr0, xr1, qr0, qr1, sr0, sr1, wb0, wb1,
                  gs0, gs1, eloc,
                  r2_sh, den_sh, sg0, sg1, ss0, ss1):
    c = lax.axis_index("c")
    s = lax.axis_index("s")
    wid = s * _NC + c
    Gp = den_sh.shape[0]
    nblk_tot = idx2_hbm.shape[0]
    D = xr0.shape[1]
    nch = D // 16
    ibs, xrs, qrs = (ib0, ib1), (xr0, xr1), (qr0, qr1)
    srs, wbs, gss = (sr0, sr1), (wb0, wb1), (gs0, gs1)
    sgs, sss = (sg0, sg1), (ss0, ss1)

    rows_per_s = Gp // _NS
    r0 = s * rows_per_s
    pltpu.sync_copy(z80.at[pl.ds(r0, rows_per_s)], r2_sh.at[pl.ds(r0, rows_per_s)])
    pltpu.sync_copy(z1.at[pl.ds(r0, rows_per_s)], den_sh.at[pl.ds(r0, rows_per_s)])
    plsc.subcore_barrier()

    m15 = lax.iota(jnp.int32, 16) == 15
    nblk_w = (nblk_tot - wid + _NW - 1) // _NW

    def start_loads(i, k):
        b = wid + i * _NW
        pltpu.sync_copy(idx2_hbm.at[b], ibs[k])
        pltpu.async_copy(xa_hbm.at[ibs[k].at[0]], xrs[k], sgs[k])
        pltpu.async_copy(q2_hbm.at[ibs[k].at[1]], qrs[k], sgs[k])

    def drain_scatters(k):
        pltpu.make_async_copy(srs[k], r2_sh.at[gss[k]], sss[k]).wait()
        pltpu.make_async_copy(wbs[k], den_sh.at[gss[k]], sss[k]).wait()

    start_loads(0, 0)

    def pair(ii, carry):
        for k in (0, 1):
            i = ii * 2 + k

            @pl.when(i < nblk_w)
            def _do(i=i, k=k):
                @pl.when(i >= 2)
                def _dr():
                    drain_scatters(k)

                pltpu.make_async_copy(xa_hbm.at[ibs[k].at[0]], xrs[k],
                                      sgs[k]).wait()
                pltpu.make_async_copy(q2_hbm.at[ibs[k].at[1]], qrs[k],
                                      sgs[k]).wait()

                @pl.when(i + 1 < nblk_w)
                def _pf():
                    start_loads(i + 1, k ^ 1)

                for t in range(_BLK // 16):
                    gss[k][pl.ds(16 * t, 16)] = ibs[k][1, pl.ds(16 * t, 16)]

                xrows, qrows, srows, wblk = xrs[k], qrs[k], srs[k], wbs[k]

                def tile_body(j, carry2):
                    for i2 in range(16):
                        row = j * 16 + i2
                        p = xrows[row, pl.ds(0, 16)] * qrows[row, pl.ds(0, 16)]
                        for ch in range(1, nch):
                            p = p + (xrows[row, pl.ds(16 * ch, 16)]
                                     * qrows[row, pl.ds(16 * ch, 16)])
                        plsc.store_compressed(eloc.at[pl.ds(i2, 16)],
                                              plsc.cumsum(p), mask=m15)
                    wv = jnp.exp(eloc[pl.ds(0, 16)])
                    wblk[pl.ds(j * 16, 16)] = wv
                    for i2 in range(16):
                        row = j * 16 + i2
                        ws = wv[i2]
                        for ch in range(nch):
                            srows[row, pl.ds(16 * ch, 16)] = (
                                xrows[row, pl.ds(16 * ch, 16)] * ws)
                    return carry2

                lax.fori_loop(0, _BLK // 16, tile_body, 0)
                pltpu.async_copy(srows, r2_sh.at[gss[k]], sss[k], add=True)
                pltpu.async_copy(wblk, den_sh.at[gss[k]], sss[k], add=True)
        return carry

    lax.fori_loop(0, (nblk_w + 1) // 2, pair, 0)
    for k in (0, 1):
        @pl.when(nblk_w > k)
        def _fin(k=k):
            drain_scatters(k)
    plsc.subcore_barrier()

    pltpu.sync_copy(r2_sh.at[pl.ds(r0, rows_per_s)],
                    r2_out.at[c, pl.ds(r0, rows_per_s)])
    pltpu.sync_copy(den_sh.at[pl.ds(r0, rows_per_s)],
                    den_out.at[c, pl.ds(r0, rows_per_s)])


def _sc_attn(idx2, xa_proj, q2, Gm):
    Dx = xa_proj.shape[1]
    Gp = ((Gm + _NS * 8 - 1) // (_NS * 8)) * (_NS * 8)
    z80 = jnp.zeros((Gp, Dx), jnp.float32)
    z1 = jnp.zeros((Gp,), jnp.float32)
    f = pl.kernel(
        _sc_attn_body,
        out_type=(
            jax.ShapeDtypeStruct((2, Gp, Dx), jnp.float32),
            jax.ShapeDtypeStruct((2, Gp), jnp.float32),
        ),
        mesh=plsc.VectorSubcoreMesh(core_axis_name="c", subcore_axis_name="s"),
        compiler_params=pltpu.CompilerParams(use_tc_tiling_on_sc=False,
                                             needs_layout_passes=False),
        scratch_types=[
            pltpu.VMEM((2, _BLK), jnp.int32),
            pltpu.VMEM((2, _BLK), jnp.int32),
            pltpu.VMEM((_BLK, Dx), jnp.float32),
            pltpu.VMEM((_BLK, Dx), jnp.float32),
            pltpu.VMEM((_BLK, Dx), jnp.float32),
            pltpu.VMEM((_BLK, Dx), jnp.float32),
            pltpu.VMEM((_BLK, Dx), jnp.float32),
            pltpu.VMEM((_BLK, Dx), jnp.float32),
            pltpu.VMEM((_BLK,), jnp.float32),
            pltpu.VMEM((_BLK,), jnp.float32),
            pltpu.VMEM((_BLK,), jnp.int32),
            pltpu.VMEM((_BLK,), jnp.int32),
            pltpu.VMEM((32,), jnp.float32),
            pltpu.VMEM_SHARED((Gp, Dx), jnp.float32),
            pltpu.VMEM_SHARED((Gp,), jnp.float32),
            pltpu.SemaphoreType.DMA,
            pltpu.SemaphoreType.DMA,
            pltpu.SemaphoreType.DMA,
            pltpu.SemaphoreType.DMA,
        ],
    )
    return f(idx2, xa_proj, q2, z80, z1)


def _sc_seg_sum(idx2, xa_proj, cond_atom, Gm):
    Dx = xa_proj.shape[1]
    Dc = cond_atom.shape[1]
    Gp = ((Gm + _NS * 8 - 1) // (_NS * 8)) * (_NS * 8)
    z80 = jnp.zeros((Gp, Dx), jnp.float32)
    z64 = jnp.zeros((Gp, Dc), jnp.float32)
    z1 = jnp.zeros((Gp,), jnp.float32)
    f = pl.kernel(
        _sc_seg_sum_body,
        out_type=(
            jax.ShapeDtypeStruct((2, Gp, Dx), jnp.float32),
            jax.ShapeDtypeStruct((2, Gp, Dc), jnp.float32),
            jax.ShapeDtypeStruct((2, Gp), jnp.float32),
        ),
        mesh=plsc.VectorSubcoreMesh(core_axis_name="c", subcore_axis_name="s"),
        compiler_params=pltpu.CompilerParams(use_tc_tiling_on_sc=False,
                                             needs_layout_passes=False),
        scratch_types=[
            pltpu.VMEM((2, _BLK), jnp.int32),
            pltpu.VMEM((2, _BLK), jnp.int32),
            pltpu.VMEM((_BLK, Dx), jnp.float32),
            pltpu.VMEM((_BLK, Dx), jnp.float32),
            pltpu.VMEM((_BLK, Dc), jnp.float32),
            pltpu.VMEM((_BLK, Dc), jnp.float32),
            pltpu.VMEM((_BLK,), jnp.int32),
            pltpu.VMEM((_BLK,), jnp.int32),
            pltpu.VMEM((_BLK,), jnp.float32),
            pltpu.VMEM_SHARED((Gp, Dx), jnp.float32),
            pltpu.VMEM_SHARED((Gp, Dc), jnp.float32),
            pltpu.VMEM_SHARED((Gp,), jnp.float32),
            pltpu.SemaphoreType.DMA,
            pltpu.SemaphoreType.DMA,
            pltpu.SemaphoreType.DMA,
            pltpu.SemaphoreType.DMA,
        ],
    )
    return f(idx2, xa_proj, cond_atom, z80, z64, z1)


def _tc_pre_body(x_atom_ref, W_a_ref, b_a_ref, x_group_ref, W_g_ref, b_g_ref,
                 xa_ref, xg0_ref, tid_ref):
    xa_ref[...] = lax.dot_general(
        x_atom_ref[...], W_a_ref[...], (((1,), (1,)), ((), ())),
        preferred_element_type=jnp.float32) + b_a_ref[...]
    Xg = x_group_ref[...][:, 0:40]
    xg0_ref[...] = lax.dot_general(
        Xg, W_g_ref[...], (((1,), (1,)), ((), ())),
        preferred_element_type=jnp.float32) + b_g_ref[...]
    m = jnp.max(Xg, axis=1, keepdims=True)
    cols = lax.broadcasted_iota(jnp.int32, Xg.shape, 1)
    tid = jnp.min(jnp.where(Xg == m, cols, jnp.int32(2**30)), axis=1)
    tid_ref[...] = tid.astype(jnp.int32)


def _tc_pre(x_atom, W_aproj, b_aproj, x_group, W_gproj, b_gproj):
    Na = x_atom.shape[0]
    Gm = x_group.shape[0]
    out = pl.pallas_call(
        _tc_pre_body,
        out_shape=(
            jax.ShapeDtypeStruct((Na, 80), jnp.float32),
            jax.ShapeDtypeStruct((Gm, 80), jnp.float32),
            jax.ShapeDtypeStruct((Gm,), jnp.int32),
        ),
    )(x_atom, W_aproj, b_aproj.reshape(1, -1), x_group, W_gproj,
      b_gproj.reshape(1, -1))
    return out


def kernel(x_atom, atom_idx, x_group, group_idx, edge_index_group, cond_atom,
           W_gproj, b_gproj, W_aproj, b_aproj, Wih, Whh, bih, bhh,
           W_merge, b_merge, Wg1, bg1, Wg2, bg2, Wb1, bb1, Wb2, bb2):
    Gm = x_group.shape[0]
    Ninc = atom_idx.shape[0]

    xa_proj, xg0, type_ids = _tc_pre(x_atom, W_aproj, b_aproj,
                                     x_group, W_gproj, b_gproj)

    idx2 = jnp.stack([atom_idx.reshape(-1, _BLK),
                      group_idx.reshape(-1, _BLK)], axis=1)
    xs, cs, cn = _sc_seg_sum(idx2, xa_proj, cond_atom, Gm)

    q2, cond_g, gamma, beta, cntc = _tc_mid(
        xs, cs, cn, Wih, Whh, bih, bhh, Wg1, bg1, Wg2, bg2,
        Wb1, bb1, Wb2, bb2, Gm)

    r2p, denp = _sc_attn(idx2, xa_proj, q2, Gm)

    xg = _tc_fin(xg0, q2, r2p, denp, cntc, gamma, beta, cond_g,
                 W_merge, b_merge)
    return (x_atom, xg, type_ids)


# speedup vs baseline: 1.6774x; 1.0571x over previous
import functools

import jax
import jax.numpy as jnp
from jax import lax
from jax.experimental import pallas as pl
from jax.experimental.pallas import tpu as pltpu
from jax.experimental.pallas import tpu_sc as plsc

_NC, _NS = 2, 16
_NW = _NC * _NS
_BLK = 128


def _sc_seg_sum_body(idx2_hbm, xa_hbm, cond_hbm, z80, z64, z1,
                     xsum_out, csum_out, cnt_out,
                     ib0, ib1, xr0, xr1, cr0, cr1, gs0, gs1, ones,
                     xsum_sh, csum_sh, cnt_sh, sg0, sg1, ss0, ss1):
    c = lax.axis_index("c")
    s = lax.axis_index("s")
    wid = s * _NC + c
    Gp = cnt_sh.shape[0]
    nblk_tot = idx2_hbm.shape[0]

    for i in range(_BLK // 16):
        ones[pl.ds(16 * i, 16)] = jnp.ones((16,), jnp.float32)

    rows_per_s = Gp // _NS
    r0 = s * rows_per_s
    pltpu.sync_copy(z80.at[pl.ds(r0, rows_per_s)], xsum_sh.at[pl.ds(r0, rows_per_s)])
    pltpu.sync_copy(z64.at[pl.ds(r0, rows_per_s)], csum_sh.at[pl.ds(r0, rows_per_s)])
    pltpu.sync_copy(z1.at[pl.ds(r0, rows_per_s)], cnt_sh.at[pl.ds(r0, rows_per_s)])
    plsc.subcore_barrier()

    nblk_w = (nblk_tot - wid + _NW - 1) // _NW
    ibs, xrs, crs = (ib0, ib1), (xr0, xr1), (cr0, cr1)
    gss, sgs, sss = (gs0, gs1), (sg0, sg1), (ss0, ss1)

    def start_loads(i, k):
        b = wid + i * _NW
        pltpu.sync_copy(idx2_hbm.at[b], ibs[k])
        pltpu.async_copy(xa_hbm.at[ibs[k].at[0]], xrs[k], sgs[k])
        pltpu.async_copy(cond_hbm.at[ibs[k].at[0]], crs[k], sgs[k])

    def drain_scatters(k):
        pltpu.make_async_copy(xrs[k], xsum_sh.at[gss[k]], sss[k]).wait()
        pltpu.make_async_copy(crs[k], csum_sh.at[gss[k]], sss[k]).wait()
        pltpu.make_async_copy(ones, cnt_sh.at[gss[k]], sss[k]).wait()

    start_loads(0, 0)

    def pair(ii, carry):
        for k in (0, 1):
            i = ii * 2 + k

            @pl.when(i < nblk_w)
            def _do(i=i, k=k):
                pltpu.make_async_copy(xa_hbm.at[ibs[k].at[0]], xrs[k],
                                      sgs[k]).wait()
                pltpu.make_async_copy(cond_hbm.at[ibs[k].at[0]], crs[k],
                                      sgs[k]).wait()
                for t in range(_BLK // 16):
                    gss[k][pl.ds(16 * t, 16)] = ibs[k][1, pl.ds(16 * t, 16)]

                @pl.when(i >= 1)
                def _dr():
                    drain_scatters(k ^ 1)

                @pl.when(i + 1 < nblk_w)
                def _pf():
                    start_loads(i + 1, k ^ 1)

                pltpu.async_copy(xrs[k], xsum_sh.at[gss[k]], sss[k], add=True)
                pltpu.async_copy(crs[k], csum_sh.at[gss[k]], sss[k], add=True)
                pltpu.async_copy(ones, cnt_sh.at[gss[k]], sss[k], add=True)
        return carry

    lax.fori_loop(0, (nblk_w + 1) // 2, pair, 0)
    for k in (0, 1):
        @pl.when((nblk_w - 1) % 2 == k)
        def _fin(k=k):
            drain_scatters(k)
    plsc.subcore_barrier()

    pltpu.sync_copy(xsum_sh.at[pl.ds(r0, rows_per_s)],
                    xsum_out.at[c, pl.ds(r0, rows_per_s)])
    pltpu.sync_copy(csum_sh.at[pl.ds(r0, rows_per_s)],
                    csum_out.at[c, pl.ds(r0, rows_per_s)])
    pltpu.sync_copy(cnt_sh.at[pl.ds(r0, rows_per_s)],
                    cnt_out.at[c, pl.ds(r0, rows_per_s)])


def _mm(x, W):
    return lax.dot_general(x, W, (((1,), (1,)), ((), ())),
                           preferred_element_type=jnp.float32)


def _sigmoid(x):
    return 1.0 / (1.0 + jnp.exp(-x))


def _tc_mid_body(xs_ref, cs_ref, cn_ref, Wih_ref, Whh_ref, bih_ref, bhh_ref,
                 Wg1_ref, bg1_ref, Wg2_ref, bg2_ref,
                 Wb1_ref, bb1_ref, Wb2_ref, bb2_ref,
                 q2_ref, cond_ref, gamma_ref, beta_ref, cnt_ref):
    Gm = q2_ref.shape[0]
    D = q2_ref.shape[1]
    cnt = cn_ref[0, :Gm] + cn_ref[1, :Gm]
    xsum = xs_ref[0, :Gm] + xs_ref[1, :Gm]
    csum = cs_ref[0, :Gm] + cs_ref[1, :Gm]
    cntc = cnt[:, None]
    r1 = xsum / (cntc + 1e-16)
    cond_g = csum / jnp.maximum(cntc, 1.0)

    bihv = bih_ref[...]
    bhhv = bhh_ref[...]
    gates1 = bihv + bhhv
    i1 = _sigmoid(gates1[:, 0:D])
    g1 = jnp.tanh(gates1[:, 2 * D:3 * D])
    o1 = _sigmoid(gates1[:, 3 * D:4 * D])
    c1 = i1 * g1
    h1 = o1 * jnp.tanh(c1)
    h1b = jnp.broadcast_to(h1, (Gm, D))
    q_star1 = jnp.concatenate([h1b, r1], axis=1)

    gates = _mm(q_star1, Wih_ref[...]) + bihv + _mm(h1b, Whh_ref[...]) + bhhv
    i2 = _sigmoid(gates[:, 0:D])
    f2 = _sigmoid(gates[:, D:2 * D])
    g2 = jnp.tanh(gates[:, 2 * D:3 * D])
    o2 = _sigmoid(gates[:, 3 * D:4 * D])
    c2 = f2 * jnp.broadcast_to(c1, (Gm, D)) + i2 * g2
    q2_ref[...] = o2 * jnp.tanh(c2)

    cond_ref[...] = cond_g
    gamma_ref[...] = _mm(jnp.maximum(_mm(cond_g, Wg1_ref[...]) + bg1_ref[...],
                                     0.0), Wg2_ref[...]) + bg2_ref[...]
    beta_ref[...] = _mm(jnp.maximum(_mm(cond_g, Wb1_ref[...]) + bb1_ref[...],
                                    0.0), Wb2_ref[...]) + bb2_ref[...]
    cnt_ref[...] = cntc


def _tc_mid(xs, cs, cn, Wih, Whh, bih, bhh, Wg1, bg1, Wg2, bg2,
            Wb1, bb1, Wb2, bb2, Gm):
    D = 80
    Dc = cs.shape[2]
    return pl.pallas_call(
        _tc_mid_body,
        out_shape=(
            jax.ShapeDtypeStruct((Gm, D), jnp.float32),
            jax.ShapeDtypeStruct((Gm, Dc), jnp.float32),
            jax.ShapeDtypeStruct((Gm, 199), jnp.float32),
            jax.ShapeDtypeStruct((Gm, 199), jnp.float32),
            jax.ShapeDtypeStruct((Gm, 1), jnp.float32),
        ),
    )(xs, cs, cn, Wih, Whh, bih.reshape(1, -1), bhh.reshape(1, -1),
      Wg1, bg1.reshape(1, -1), Wg2, bg2.reshape(1, -1),
      Wb1, bb1.reshape(1, -1), Wb2, bb2.reshape(1, -1))


def _tc_fin_body(xg0_ref, q2_ref, r2_ref, den_ref, cnt_ref, gamma_ref,
                 beta_ref, cond_ref, Wm_ref, bm_ref, out_ref):
    Gm = xg0_ref.shape[0]
    r2 = ((r2_ref[0, :Gm] + r2_ref[1, :Gm])
          / (den_ref[0, :Gm] + den_ref[1, :Gm] + 1e-16)[:, None])
    present = cnt_ref[...] > 0.0
    q_star = jnp.where(present, jnp.concatenate([q2_ref[...], r2], axis=1), 0.0)
    xg = _mm(jnp.concatenate([xg0_ref[...], q_star], axis=1), Wm_ref[...]) \
        + bm_ref[...]
    xg = gamma_ref[...] * xg + beta_ref[...]
    out_ref[...] = jnp.concatenate([xg, cond_ref[...]], axis=1)


def _tc_fin(xg0, q2, r2p, denp, cnt, gamma, beta, cond_g, W_merge, b_merge):
    Gm = xg0.shape[0]
    return pl.pallas_call(
        _tc_fin_body,
        out_shape=jax.ShapeDtypeStruct((Gm, 263), jnp.float32),
    )(xg0, q2, r2p, denp, cnt, gamma, beta, cond_g, W_merge,
      b_merge.reshape(1, -1))


def _sc_attn_body(idx2_hbm, xa_hbm, q2_hbm, z80, z1,
                  r2_out, den_out,
                  ib0, ib1, xr0, xr1, qr0, qr1, sr0, sr1, wb0, wb1,
                  gs0, gs1, eloc,
                  r2_sh, den_sh, sg0, sg1, ss0, ss1, si0, si1):
    c = lax.axis_index("c")
    s = lax.axis_index("s")
    wid = s * _NC + c
    Gp = den_sh.shape[0]
    nblk_tot = idx2_hbm.shape[0]
    D = xr0.shape[1]
    nch = D // 16
    ibs, xrs, qrs = (ib0, ib1), (xr0, xr1), (qr0, qr1)
    srs, wbs, gss = (sr0, sr1), (wb0, wb1), (gs0, gs1)
    sgs, sss, sis = (sg0, sg1), (ss0, ss1), (si0, si1)

    rows_per_s = Gp // _NS
    r0 = s * rows_per_s
    pltpu.sync_copy(z80.at[pl.ds(r0, rows_per_s)], r2_sh.at[pl.ds(r0, rows_per_s)])
    pltpu.sync_copy(z1.at[pl.ds(r0, rows_per_s)], den_sh.at[pl.ds(r0, rows_per_s)])
    plsc.subcore_barrier()

    m15 = lax.iota(jnp.int32, 16) == 15
    nblk_w = (nblk_tot - wid + _NW - 1) // _NW

    def start_idx(i, k):
        pltpu.async_copy(idx2_hbm.at[wid + i * _NW], ibs[k], sis[k])

    def wait_idx(i, k):
        pltpu.make_async_copy(idx2_hbm.at[wid + i * _NW], ibs[k],
                              sis[k]).wait()

    def start_gathers(k):
        pltpu.async_copy(xa_hbm.at[ibs[k].at[0]], xrs[k], sgs[k])
        pltpu.async_copy(q2_hbm.at[ibs[k].at[1]], qrs[k], sgs[k])

    def drain_scatters(k):
        pltpu.make_async_copy(srs[k], r2_sh.at[gss[k]], sss[k]).wait()
        pltpu.make_async_copy(wbs[k], den_sh.at[gss[k]], sss[k]).wait()

    start_idx(0, 0)
    wait_idx(0, 0)
    start_gathers(0)

    @pl.when(nblk_w > 1)
    def _pro():
        start_idx(1, 1)

    def pair(ii, carry):
        for k in (0, 1):
            i = ii * 2 + k

            @pl.when(i < nblk_w)
            def _do(i=i, k=k):
                @pl.when(i >= 2)
                def _dr():
                    drain_scatters(k)

                pltpu.make_async_copy(xa_hbm.at[ibs[k].at[0]], xrs[k],
                                      sgs[k]).wait()
                pltpu.make_async_copy(q2_hbm.at[ibs[k].at[1]], qrs[k],
                                      sgs[k]).wait()

                for t in range(_BLK // 16):
                    gss[k][pl.ds(16 * t, 16)] = ibs[k][1, pl.ds(16 * t, 16)]

                @pl.when(i + 1 < nblk_w)
                def _pf():
                    wait_idx(i + 1, k ^ 1)
                    start_gathers(k ^ 1)

                @pl.when(i + 2 < nblk_w)
                def _pf2():
                    start_idx(i + 2, k)

                xrows, qrows, srows, wblk = xrs[k], qrs[k], srs[k], wbs[k]

                def tile_body(j, carry2):
                    for i2 in range(16):
                        row = j * 16 + i2
                        p = xrows[row, pl.ds(0, 16)] * qrows[row, pl.ds(0, 16)]
                        for ch in range(1, nch):
                            p = p + (xrows[row, pl.ds(16 * ch, 16)]
                                     * qrows[row, pl.ds(16 * ch, 16)])
                        plsc.store_compressed(eloc.at[pl.ds(i2, 16)],
                                              plsc.cumsum(p), mask=m15)
                    wv = jnp.exp(eloc[pl.ds(0, 16)])
                    wblk[pl.ds(j * 16, 16)] = wv
                    for i2 in range(16):
                        row = j * 16 + i2
                        ws = wv[i2]
                        for ch in range(nch):
                            srows[row, pl.ds(16 * ch, 16)] = (
                                xrows[row, pl.ds(16 * ch, 16)] * ws)
                    return carry2

                lax.fori_loop(0, _BLK // 16, tile_body, 0)
                pltpu.async_copy(srows, r2_sh.at[gss[k]], sss[k], add=True)
                pltpu.async_copy(wblk, den_sh.at[gss[k]], sss[k], add=True)
        return carry

    lax.fori_loop(0, (nblk_w + 1) // 2, pair, 0)
    for k in (0, 1):
        @pl.when(nblk_w > k)
        def _fin(k=k):
            drain_scatters(k)
    plsc.subcore_barrier()

    pltpu.sync_copy(r2_sh.at[pl.ds(r0, rows_per_s)],
                    r2_out.at[c, pl.ds(r0, rows_per_s)])
    pltpu.sync_copy(den_sh.at[pl.ds(r0, rows_per_s)],
                    den_out.at[c, pl.ds(r0, rows_per_s)])


def _sc_attn(idx2, xa_proj, q2, Gm):
    Dx = xa_proj.shape[1]
    Gp = ((Gm + _NS * 8 - 1) // (_NS * 8)) * (_NS * 8)
    z80 = jnp.zeros((Gp, Dx), jnp.float32)
    z1 = jnp.zeros((Gp,), jnp.float32)
    f = pl.kernel(
        _sc_attn_body,
        out_type=(
            jax.ShapeDtypeStruct((2, Gp, Dx), jnp.float32),
            jax.ShapeDtypeStruct((2, Gp), jnp.float32),
        ),
        mesh=plsc.VectorSubcoreMesh(core_axis_name="c", subcore_axis_name="s"),
        compiler_params=pltpu.CompilerParams(use_tc_tiling_on_sc=False,
                                             needs_layout_passes=False),
        scratch_types=[
            pltpu.VMEM((2, _BLK), jnp.int32),
            pltpu.VMEM((2, _BLK), jnp.int32),
            pltpu.VMEM((_BLK, Dx), jnp.float32),
            pltpu.VMEM((_BLK, Dx), jnp.float32),
            pltpu.VMEM((_BLK, Dx), jnp.float32),
            pltpu.VMEM((_BLK, Dx), jnp.float32),
            pltpu.VMEM((_BLK, Dx), jnp.float32),
            pltpu.VMEM((_BLK, Dx), jnp.float32),
            pltpu.VMEM((_BLK,), jnp.float32),
            pltpu.VMEM((_BLK,), jnp.float32),
            pltpu.VMEM((_BLK,), jnp.int32),
            pltpu.VMEM((_BLK,), jnp.int32),
            pltpu.VMEM((32,), jnp.float32),
            pltpu.VMEM_SHARED((Gp, Dx), jnp.float32),
            pltpu.VMEM_SHARED((Gp,), jnp.float32),
            pltpu.SemaphoreType.DMA,
            pltpu.SemaphoreType.DMA,
            pltpu.SemaphoreType.DMA,
            pltpu.SemaphoreType.DMA,
            pltpu.SemaphoreType.DMA,
            pltpu.SemaphoreType.DMA,
        ],
    )
    return f(idx2, xa_proj, q2, z80, z1)


def _sc_seg_sum(idx2, xa_proj, cond_atom, Gm):
    Dx = xa_proj.shape[1]
    Dc = cond_atom.shape[1]
    Gp = ((Gm + _NS * 8 - 1) // (_NS * 8)) * (_NS * 8)
    z80 = jnp.zeros((Gp, Dx), jnp.float32)
    z64 = jnp.zeros((Gp, Dc), jnp.float32)
    z1 = jnp.zeros((Gp,), jnp.float32)
    f = pl.kernel(
        _sc_seg_sum_body,
        out_type=(
            jax.ShapeDtypeStruct((2, Gp, Dx), jnp.float32),
            jax.ShapeDtypeStruct((2, Gp, Dc), jnp.float32),
            jax.ShapeDtypeStruct((2, Gp), jnp.float32),
        ),
        mesh=plsc.VectorSubcoreMesh(core_axis_name="c", subcore_axis_name="s"),
        compiler_params=pltpu.CompilerParams(use_tc_tiling_on_sc=False,
                                             needs_layout_passes=False),
        scratch_types=[
            pltpu.VMEM((2, _BLK), jnp.int32),
            pltpu.VMEM((2, _BLK), jnp.int32),
            pltpu.VMEM((_BLK, Dx), jnp.float32),
            pltpu.VMEM((_BLK, Dx), jnp.float32),
            pltpu.VMEM((_BLK, Dc), jnp.float32),
            pltpu.VMEM((_BLK, Dc), jnp.float32),
            pltpu.VMEM((_BLK,), jnp.int32),
            pltpu.VMEM((_BLK,), jnp.int32),
            pltpu.VMEM((_BLK,), jnp.float32),
            pltpu.VMEM_SHARED((Gp, Dx), jnp.float32),
            pltpu.VMEM_SHARED((Gp, Dc), jnp.float32),
            pltpu.VMEM_SHARED((Gp,), jnp.float32),
            pltpu.SemaphoreType.DMA,
            pltpu.SemaphoreType.DMA,
            pltpu.SemaphoreType.DMA,
            pltpu.SemaphoreType.DMA,
        ],
    )
    return f(idx2, xa_proj, cond_atom, z80, z64, z1)


def _tc_pre_body(x_atom_ref, W_a_ref, b_a_ref, x_group_ref, W_g_ref, b_g_ref,
                 xa_ref, xg0_ref, tid_ref):
    xa_ref[...] = lax.dot_general(
        x_atom_ref[...], W_a_ref[...], (((1,), (1,)), ((), ())),
        preferred_element_type=jnp.float32) + b_a_ref[...]
    Xg = x_group_ref[...][:, 0:40]
    xg0_ref[...] = lax.dot_general(
        Xg, W_g_ref[...], (((1,), (1,)), ((), ())),
        preferred_element_type=jnp.float32) + b_g_ref[...]
    m = jnp.max(Xg, axis=1, keepdims=True)
    cols = lax.broadcasted_iota(jnp.int32, Xg.shape, 1)
    tid = jnp.min(jnp.where(Xg == m, cols, jnp.int32(2**30)), axis=1)
    tid_ref[...] = tid.astype(jnp.int32)


def _tc_pre(x_atom, W_aproj, b_aproj, x_group, W_gproj, b_gproj):
    Na = x_atom.shape[0]
    Gm = x_group.shape[0]
    out = pl.pallas_call(
        _tc_pre_body,
        out_shape=(
            jax.ShapeDtypeStruct((Na, 80), jnp.float32),
            jax.ShapeDtypeStruct((Gm, 80), jnp.float32),
            jax.ShapeDtypeStruct((Gm,), jnp.int32),
        ),
    )(x_atom, W_aproj, b_aproj.reshape(1, -1), x_group, W_gproj,
      b_gproj.reshape(1, -1))
    return out


def kernel(x_atom, atom_idx, x_group, group_idx, edge_index_group, cond_atom,
           W_gproj, b_gproj, W_aproj, b_aproj, Wih, Whh, bih, bhh,
           W_merge, b_merge, Wg1, bg1, Wg2, bg2, Wb1, bb1, Wb2, bb2):
    Gm = x_group.shape[0]
    Ninc = atom_idx.shape[0]

    xa_proj, xg0, type_ids = _tc_pre(x_atom, W_aproj, b_aproj,
                                     x_group, W_gproj, b_gproj)

    idx2 = jnp.stack([atom_idx.reshape(-1, _BLK),
                      group_idx.reshape(-1, _BLK)], axis=1)
    xs, cs, cn = _sc_seg_sum(idx2, xa_proj, cond_atom, Gm)

    q2, cond_g, gamma, beta, cntc = _tc_mid(
        xs, cs, cn, Wih, Whh, bih, bhh, Wg1, bg1, Wg2, bg2,
        Wb1, bb1, Wb2, bb2, Gm)

    r2p, denp = _sc_attn(idx2, xa_proj, q2, Gm)

    xg = _tc_fin(xg0, q2, r2p, denp, cntc, gamma, beta, cond_g,
                 W_merge, b_merge)
    return (x_atom, xg, type_ids)


# speedup vs baseline: 1.7555x; 1.0466x over previous
import functools

import jax
import jax.numpy as jnp
from jax import lax
from jax.experimental import pallas as pl
from jax.experimental.pallas import tpu as pltpu
from jax.experimental.pallas import tpu_sc as plsc

_NC, _NS = 2, 16
_NW = _NC * _NS
_BLK = 128


def _sc_seg_sum_body(idx2_hbm, xa_hbm, cond_hbm, z80, z64, z1,
                     xsum_out, csum_out, cnt_out,
                     ib0, ib1, xr0, xr1, cr0, cr1, gs0, gs1, ones,
                     xsum_sh, csum_sh, cnt_sh, sg0, sg1, ss0, ss1, si0, si1):
    c = lax.axis_index("c")
    s = lax.axis_index("s")
    wid = s * _NC + c
    Gp = cnt_sh.shape[0]
    nblk_tot = idx2_hbm.shape[0]

    for i in range(_BLK // 16):
        ones[pl.ds(16 * i, 16)] = jnp.ones((16,), jnp.float32)

    rows_per_s = Gp // _NS
    r0 = s * rows_per_s
    pltpu.sync_copy(z80.at[pl.ds(r0, rows_per_s)], xsum_sh.at[pl.ds(r0, rows_per_s)])
    pltpu.sync_copy(z64.at[pl.ds(r0, rows_per_s)], csum_sh.at[pl.ds(r0, rows_per_s)])
    pltpu.sync_copy(z1.at[pl.ds(r0, rows_per_s)], cnt_sh.at[pl.ds(r0, rows_per_s)])
    plsc.subcore_barrier()

    nblk_w = (nblk_tot - wid + _NW - 1) // _NW
    ibs, xrs, crs = (ib0, ib1), (xr0, xr1), (cr0, cr1)
    gss, sgs = (gs0, gs1), (sg0, sg1)
    sss, sis = (ss0, ss1), (si0, si1)

    def start_idx(i, k):
        pltpu.async_copy(idx2_hbm.at[wid + i * _NW], ibs[k], sis[k])

    def wait_idx(i, k):
        pltpu.make_async_copy(idx2_hbm.at[wid + i * _NW], ibs[k],
                              sis[k]).wait()

    def start_gathers(k):
        pltpu.async_copy(xa_hbm.at[ibs[k].at[0]], xrs[k], sgs[k])
        pltpu.async_copy(cond_hbm.at[ibs[k].at[0]], crs[k], sgs[k])

    def drain_scatters(k):
        pltpu.make_async_copy(xrs[k], xsum_sh.at[gss[k]], sss[k]).wait()
        pltpu.make_async_copy(crs[k], csum_sh.at[gss[k]], sss[k]).wait()
        pltpu.make_async_copy(ones, cnt_sh.at[gss[k]], sss[k]).wait()

    start_idx(0, 0)
    wait_idx(0, 0)
    start_gathers(0)

    @pl.when(nblk_w > 1)
    def _pro():
        start_idx(1, 1)

    def pair(ii, carry):
        for k in (0, 1):
            i = ii * 2 + k

            @pl.when(i < nblk_w)
            def _do(i=i, k=k):
                pltpu.make_async_copy(xa_hbm.at[ibs[k].at[0]], xrs[k],
                                      sgs[k]).wait()
                pltpu.make_async_copy(cond_hbm.at[ibs[k].at[0]], crs[k],
                                      sgs[k]).wait()
                for t in range(_BLK // 16):
                    gss[k][pl.ds(16 * t, 16)] = ibs[k][1, pl.ds(16 * t, 16)]

                @pl.when(i >= 1)
                def _dr():
                    drain_scatters(k ^ 1)

                @pl.when(i + 1 < nblk_w)
                def _pf():
                    wait_idx(i + 1, k ^ 1)
                    start_gathers(k ^ 1)

                @pl.when(i + 2 < nblk_w)
                def _pf2():
                    start_idx(i + 2, k)

                pltpu.async_copy(xrs[k], xsum_sh.at[gss[k]], sss[k], add=True)
                pltpu.async_copy(crs[k], csum_sh.at[gss[k]], sss[k], add=True)
                pltpu.async_copy(ones, cnt_sh.at[gss[k]], sss[k], add=True)
        return carry

    lax.fori_loop(0, (nblk_w + 1) // 2, pair, 0)
    for k in (0, 1):
        @pl.when((nblk_w - 1) % 2 == k)
        def _fin(k=k):
            drain_scatters(k)
    plsc.subcore_barrier()

    pltpu.sync_copy(xsum_sh.at[pl.ds(r0, rows_per_s)],
                    xsum_out.at[c, pl.ds(r0, rows_per_s)])
    pltpu.sync_copy(csum_sh.at[pl.ds(r0, rows_per_s)],
                    csum_out.at[c, pl.ds(r0, rows_per_s)])
    pltpu.sync_copy(cnt_sh.at[pl.ds(r0, rows_per_s)],
                    cnt_out.at[c, pl.ds(r0, rows_per_s)])


def _mm(x, W):
    return lax.dot_general(x, W, (((1,), (1,)), ((), ())),
                           preferred_element_type=jnp.float32)


def _sigmoid(x):
    return 1.0 / (1.0 + jnp.exp(-x))


def _tc_mid_body(xs_ref, cs_ref, cn_ref, Wih_ref, Whh_ref, bih_ref, bhh_ref,
                 Wg1_ref, bg1_ref, Wg2_ref, bg2_ref,
                 Wb1_ref, bb1_ref, Wb2_ref, bb2_ref,
                 q2_ref, cond_ref, gamma_ref, beta_ref, cnt_ref):
    Gm = q2_ref.shape[0]
    D = q2_ref.shape[1]
    cnt = cn_ref[0, :Gm] + cn_ref[1, :Gm]
    xsum = xs_ref[0, :Gm] + xs_ref[1, :Gm]
    csum = cs_ref[0, :Gm] + cs_ref[1, :Gm]
    cntc = cnt[:, None]
    r1 = xsum / (cntc + 1e-16)
    cond_g = csum / jnp.maximum(cntc, 1.0)

    bihv = bih_ref[...]
    bhhv = bhh_ref[...]
    gates1 = bihv + bhhv
    i1 = _sigmoid(gates1[:, 0:D])
    g1 = jnp.tanh(gates1[:, 2 * D:3 * D])
    o1 = _sigmoid(gates1[:, 3 * D:4 * D])
    c1 = i1 * g1
    h1 = o1 * jnp.tanh(c1)
    h1b = jnp.broadcast_to(h1, (Gm, D))
    q_star1 = jnp.concatenate([h1b, r1], axis=1)

    gates = _mm(q_star1, Wih_ref[...]) + bihv + _mm(h1b, Whh_ref[...]) + bhhv
    i2 = _sigmoid(gates[:, 0:D])
    f2 = _sigmoid(gates[:, D:2 * D])
    g2 = jnp.tanh(gates[:, 2 * D:3 * D])
    o2 = _sigmoid(gates[:, 3 * D:4 * D])
    c2 = f2 * jnp.broadcast_to(c1, (Gm, D)) + i2 * g2
    q2_ref[...] = o2 * jnp.tanh(c2)

    cond_ref[...] = cond_g
    gamma_ref[...] = _mm(jnp.maximum(_mm(cond_g, Wg1_ref[...]) + bg1_ref[...],
                                     0.0), Wg2_ref[...]) + bg2_ref[...]
    beta_ref[...] = _mm(jnp.maximum(_mm(cond_g, Wb1_ref[...]) + bb1_ref[...],
                                    0.0), Wb2_ref[...]) + bb2_ref[...]
    cnt_ref[...] = cntc


def _tc_mid(xs, cs, cn, Wih, Whh, bih, bhh, Wg1, bg1, Wg2, bg2,
            Wb1, bb1, Wb2, bb2, Gm):
    D = 80
    Dc = cs.shape[2]
    return pl.pallas_call(
        _tc_mid_body,
        out_shape=(
            jax.ShapeDtypeStruct((Gm, D), jnp.float32),
            jax.ShapeDtypeStruct((Gm, Dc), jnp.float32),
            jax.ShapeDtypeStruct((Gm, 199), jnp.float32),
            jax.ShapeDtypeStruct((Gm, 199), jnp.float32),
            jax.ShapeDtypeStruct((Gm, 1), jnp.float32),
        ),
    )(xs, cs, cn, Wih, Whh, bih.reshape(1, -1), bhh.reshape(1, -1),
      Wg1, bg1.reshape(1, -1), Wg2, bg2.reshape(1, -1),
      Wb1, bb1.reshape(1, -1), Wb2, bb2.reshape(1, -1))


def _tc_fin_body(xg0_ref, q2_ref, r2_ref, den_ref, cnt_ref, gamma_ref,
                 beta_ref, cond_ref, Wm_ref, bm_ref, out_ref):
    Gm = xg0_ref.shape[0]
    r2 = ((r2_ref[0, :Gm] + r2_ref[1, :Gm])
          / (den_ref[0, :Gm] + den_ref[1, :Gm] + 1e-16)[:, None])
    present = cnt_ref[...] > 0.0
    q_star = jnp.where(present, jnp.concatenate([q2_ref[...], r2], axis=1), 0.0)
    xg = _mm(jnp.concatenate([xg0_ref[...], q_star], axis=1), Wm_ref[...]) \
        + bm_ref[...]
    xg = gamma_ref[...] * xg + beta_ref[...]
    out_ref[...] = jnp.concatenate([xg, cond_ref[...]], axis=1)


def _tc_fin(xg0, q2, r2p, denp, cnt, gamma, beta, cond_g, W_merge, b_merge):
    Gm = xg0.shape[0]
    return pl.pallas_call(
        _tc_fin_body,
        out_shape=jax.ShapeDtypeStruct((Gm, 263), jnp.float32),
    )(xg0, q2, r2p, denp, cnt, gamma, beta, cond_g, W_merge,
      b_merge.reshape(1, -1))


def _sc_attn_body(idx2_hbm, xa_hbm, q2_hbm, z80, z1,
                  r2_out, den_out,
                  ib0, ib1, xr0, xr1, qr0, qr1, sr0, sr1, wb0, wb1,
                  gs0, gs1, eloc,
                  r2_sh, den_sh, sg0, sg1, ss0, ss1, si0, si1):
    c = lax.axis_index("c")
    s = lax.axis_index("s")
    wid = s * _NC + c
    Gp = den_sh.shape[0]
    nblk_tot = idx2_hbm.shape[0]
    D = xr0.shape[1]
    nch = D // 16
    ibs, xrs, qrs = (ib0, ib1), (xr0, xr1), (qr0, qr1)
    srs, wbs, gss = (sr0, sr1), (wb0, wb1), (gs0, gs1)
    sgs, sss, sis = (sg0, sg1), (ss0, ss1), (si0, si1)

    rows_per_s = Gp // _NS
    r0 = s * rows_per_s
    pltpu.sync_copy(z80.at[pl.ds(r0, rows_per_s)], r2_sh.at[pl.ds(r0, rows_per_s)])
    pltpu.sync_copy(z1.at[pl.ds(r0, rows_per_s)], den_sh.at[pl.ds(r0, rows_per_s)])
    plsc.subcore_barrier()

    m15 = lax.iota(jnp.int32, 16) == 15
    nblk_w = (nblk_tot - wid + _NW - 1) // _NW

    def start_idx(i, k):
        pltpu.async_copy(idx2_hbm.at[wid + i * _NW], ibs[k], sis[k])

    def wait_idx(i, k):
        pltpu.make_async_copy(idx2_hbm.at[wid + i * _NW], ibs[k],
                              sis[k]).wait()

    def start_gathers(k):
        pltpu.async_copy(xa_hbm.at[ibs[k].at[0]], xrs[k], sgs[k])
        pltpu.async_copy(q2_hbm.at[ibs[k].at[1]], qrs[k], sgs[k])

    def drain_scatters(k):
        pltpu.make_async_copy(srs[k], r2_sh.at[gss[k]], sss[k]).wait()
        pltpu.make_async_copy(wbs[k], den_sh.at[gss[k]], sss[k]).wait()

    start_idx(0, 0)
    wait_idx(0, 0)
    start_gathers(0)

    @pl.when(nblk_w > 1)
    def _pro():
        start_idx(1, 1)

    def pair(ii, carry):
        for k in (0, 1):
            i = ii * 2 + k

            @pl.when(i < nblk_w)
            def _do(i=i, k=k):
                @pl.when(i >= 2)
                def _dr():
                    drain_scatters(k)

                pltpu.make_async_copy(xa_hbm.at[ibs[k].at[0]], xrs[k],
                                      sgs[k]).wait()
                pltpu.make_async_copy(q2_hbm.at[ibs[k].at[1]], qrs[k],
                                      sgs[k]).wait()

                for t in range(_BLK // 16):
                    gss[k][pl.ds(16 * t, 16)] = ibs[k][1, pl.ds(16 * t, 16)]

                @pl.when(i + 1 < nblk_w)
                def _pf():
                    wait_idx(i + 1, k ^ 1)
                    start_gathers(k ^ 1)

                @pl.when(i + 2 < nblk_w)
                def _pf2():
                    start_idx(i + 2, k)

                xrows, qrows, srows, wblk = xrs[k], qrs[k], srs[k], wbs[k]

                def tile_body(j, carry2):
                    for i2 in range(16):
                        row = j * 16 + i2
                        p = xrows[row, pl.ds(0, 16)] * qrows[row, pl.ds(0, 16)]
                        for ch in range(1, nch):
                            p = p + (xrows[row, pl.ds(16 * ch, 16)]
                                     * qrows[row, pl.ds(16 * ch, 16)])
                        plsc.store_compressed(eloc.at[pl.ds(i2, 16)],
                                              plsc.cumsum(p), mask=m15)
                    wv = jnp.exp(eloc[pl.ds(0, 16)])
                    wblk[pl.ds(j * 16, 16)] = wv
                    for i2 in range(16):
                        row = j * 16 + i2
                        ws = wv[i2]
                        for ch in range(nch):
                            srows[row, pl.ds(16 * ch, 16)] = (
                                xrows[row, pl.ds(16 * ch, 16)] * ws)
                    return carry2

                lax.fori_loop(0, _BLK // 16, tile_body, 0)
                pltpu.async_copy(srows, r2_sh.at[gss[k]], sss[k], add=True)
                pltpu.async_copy(wblk, den_sh.at[gss[k]], sss[k], add=True)
        return carry

    lax.fori_loop(0, (nblk_w + 1) // 2, pair, 0)
    for k in (0, 1):
        @pl.when(nblk_w > k)
        def _fin(k=k):
            drain_scatters(k)
    plsc.subcore_barrier()

    pltpu.sync_copy(r2_sh.at[pl.ds(r0, rows_per_s)],
                    r2_out.at[c, pl.ds(r0, rows_per_s)])
    pltpu.sync_copy(den_sh.at[pl.ds(r0, rows_per_s)],
                    den_out.at[c, pl.ds(r0, rows_per_s)])


def _sc_attn(idx2, xa_proj, q2, Gm):
    Dx = xa_proj.shape[1]
    Gp = ((Gm + _NS * 8 - 1) // (_NS * 8)) * (_NS * 8)
    z80 = jnp.zeros((Gp, Dx), jnp.float32)
    z1 = jnp.zeros((Gp,), jnp.float32)
    f = pl.kernel(
        _sc_attn_body,
        out_type=(
            jax.ShapeDtypeStruct((2, Gp, Dx), jnp.float32),
            jax.ShapeDtypeStruct((2, Gp), jnp.float32),
        ),
        mesh=plsc.VectorSubcoreMesh(core_axis_name="c", subcore_axis_name="s"),
        compiler_params=pltpu.CompilerParams(use_tc_tiling_on_sc=False,
                                             needs_layout_passes=False),
        scratch_types=[
            pltpu.VMEM((2, _BLK), jnp.int32),
            pltpu.VMEM((2, _BLK), jnp.int32),
            pltpu.VMEM((_BLK, Dx), jnp.float32),
            pltpu.VMEM((_BLK, Dx), jnp.float32),
            pltpu.VMEM((_BLK, Dx), jnp.float32),
            pltpu.VMEM((_BLK, Dx), jnp.float32),
            pltpu.VMEM((_BLK, Dx), jnp.float32),
            pltpu.VMEM((_BLK, Dx), jnp.float32),
            pltpu.VMEM((_BLK,), jnp.float32),
            pltpu.VMEM((_BLK,), jnp.float32),
            pltpu.VMEM((_BLK,), jnp.int32),
            pltpu.VMEM((_BLK,), jnp.int32),
            pltpu.VMEM((32,), jnp.float32),
            pltpu.VMEM_SHARED((Gp, Dx), jnp.float32),
            pltpu.VMEM_SHARED((Gp,), jnp.float32),
            pltpu.SemaphoreType.DMA,
            pltpu.SemaphoreType.DMA,
            pltpu.SemaphoreType.DMA,
            pltpu.SemaphoreType.DMA,
            pltpu.SemaphoreType.DMA,
            pltpu.SemaphoreType.DMA,
        ],
    )
    return f(idx2, xa_proj, q2, z80, z1)


def _sc_seg_sum(idx2, xa_proj, cond_atom, Gm):
    Dx = xa_proj.shape[1]
    Dc = cond_atom.shape[1]
    Gp = ((Gm + _NS * 8 - 1) // (_NS * 8)) * (_NS * 8)
    z80 = jnp.zeros((Gp, Dx), jnp.float32)
    z64 = jnp.zeros((Gp, Dc), jnp.float32)
    z1 = jnp.zeros((Gp,), jnp.float32)
    f = pl.kernel(
        _sc_seg_sum_body,
        out_type=(
            jax.ShapeDtypeStruct((2, Gp, Dx), jnp.float32),
            jax.ShapeDtypeStruct((2, Gp, Dc), jnp.float32),
            jax.ShapeDtypeStruct((2, Gp), jnp.float32),
        ),
        mesh=plsc.VectorSubcoreMesh(core_axis_name="c", subcore_axis_name="s"),
        compiler_params=pltpu.CompilerParams(use_tc_tiling_on_sc=False,
                                             needs_layout_passes=False),
        scratch_types=[
            pltpu.VMEM((2, _BLK), jnp.int32),
            pltpu.VMEM((2, _BLK), jnp.int32),
            pltpu.VMEM((_BLK, Dx), jnp.float32),
            pltpu.VMEM((_BLK, Dx), jnp.float32),
            pltpu.VMEM((_BLK, Dc), jnp.float32),
            pltpu.VMEM((_BLK, Dc), jnp.float32),
            pltpu.VMEM((_BLK,), jnp.int32),
            pltpu.VMEM((_BLK,), jnp.int32),
            pltpu.VMEM((_BLK,), jnp.float32),
            pltpu.VMEM_SHARED((Gp, Dx), jnp.float32),
            pltpu.VMEM_SHARED((Gp, Dc), jnp.float32),
            pltpu.VMEM_SHARED((Gp,), jnp.float32),
            pltpu.SemaphoreType.DMA,
            pltpu.SemaphoreType.DMA,
            pltpu.SemaphoreType.DMA,
            pltpu.SemaphoreType.DMA,
            pltpu.SemaphoreType.DMA,
            pltpu.SemaphoreType.DMA,
        ],
    )
    return f(idx2, xa_proj, cond_atom, z80, z64, z1)


def _tc_pre_body(x_atom_ref, W_a_ref, b_a_ref, x_group_ref, W_g_ref, b_g_ref,
                 xa_ref, xg0_ref, tid_ref):
    xa_ref[...] = lax.dot_general(
        x_atom_ref[...], W_a_ref[...], (((1,), (1,)), ((), ())),
        preferred_element_type=jnp.float32) + b_a_ref[...]
    Xg = x_group_ref[...][:, 0:40]
    xg0_ref[...] = lax.dot_general(
        Xg, W_g_ref[...], (((1,), (1,)), ((), ())),
        preferred_element_type=jnp.float32) + b_g_ref[...]
    m = jnp.max(Xg, axis=1, keepdims=True)
    cols = lax.broadcasted_iota(jnp.int32, Xg.shape, 1)
    tid = jnp.min(jnp.where(Xg == m, cols, jnp.int32(2**30)), axis=1)
    tid_ref[...] = tid.astype(jnp.int32)


def _tc_pre(x_atom, W_aproj, b_aproj, x_group, W_gproj, b_gproj):
    Na = x_atom.shape[0]
    Gm = x_group.shape[0]
    out = pl.pallas_call(
        _tc_pre_body,
        out_shape=(
            jax.ShapeDtypeStruct((Na, 80), jnp.float32),
            jax.ShapeDtypeStruct((Gm, 80), jnp.float32),
            jax.ShapeDtypeStruct((Gm,), jnp.int32),
        ),
    )(x_atom, W_aproj, b_aproj.reshape(1, -1), x_group, W_gproj,
      b_gproj.reshape(1, -1))
    return out


def kernel(x_atom, atom_idx, x_group, group_idx, edge_index_group, cond_atom,
           W_gproj, b_gproj, W_aproj, b_aproj, Wih, Whh, bih, bhh,
           W_merge, b_merge, Wg1, bg1, Wg2, bg2, Wb1, bb1, Wb2, bb2):
    Gm = x_group.shape[0]
    Ninc = atom_idx.shape[0]

    xa_proj, xg0, type_ids = _tc_pre(x_atom, W_aproj, b_aproj,
                                     x_group, W_gproj, b_gproj)

    idx2 = jnp.stack([atom_idx.reshape(-1, _BLK),
                      group_idx.reshape(-1, _BLK)], axis=1)
    xs, cs, cn = _sc_seg_sum(idx2, xa_proj, cond_atom, Gm)

    q2, cond_g, gamma, beta, cntc = _tc_mid(
        xs, cs, cn, Wih, Whh, bih, bhh, Wg1, bg1, Wg2, bg2,
        Wb1, bb1, Wb2, bb2, Gm)

    r2p, denp = _sc_attn(idx2, xa_proj, q2, Gm)

    xg = _tc_fin(xg0, q2, r2p, denp, cntc, gamma, beta, cond_g,
                 W_merge, b_merge)
    return (x_atom, xg, type_ids)
